# baseline (speedup 1.0000x reference)
import functools

import jax
import jax.numpy as jnp
import numpy as np
from jax.experimental import pallas as pl
from jax.experimental.pallas import tpu as pltpu

_N_LAYERS = 6
_H = 64


def _mm_kernel(x_ref, w_ref, b_ref, o_ref):
    o_ref[...] = (
        jnp.dot(x_ref[...], w_ref[...], preferred_element_type=jnp.float32)
        + b_ref[...]
    )


def _matmul_bias(x, w, b, bm=2000):
    m, k = x.shape
    _, n = w.shape
    assert m % bm == 0, (m, bm)
    b2 = b.reshape(1, n)
    return pl.pallas_call(
        _mm_kernel,
        grid=(m // bm,),
        in_specs=[
            pl.BlockSpec((bm, k), lambda i: (i, 0)),
            pl.BlockSpec((k, n), lambda i: (0, 0)),
            pl.BlockSpec((1, n), lambda i: (0, 0)),
        ],
        out_specs=pl.BlockSpec((bm, n), lambda i: (i, 0)),
        out_shape=jax.ShapeDtypeStruct((m, n), jnp.float32),
    )(x, w, b2)


def _seg_softmax(alpha, idx, n):
    m = jax.ops.segment_max(alpha, idx, num_segments=n)
    m = jnp.where(jnp.isfinite(m), m, 0.0)
    ex = jnp.exp(alpha - m[idx])
    s = jax.ops.segment_sum(ex, idx, num_segments=n)
    return ex / (s[idx] + 1e-16)


def kernel(x, edge_index, edge_attr, batch, non_edge_index, params):
    p = params
    n = x.shape[0]
    g = 128
    src, dst = edge_index[0], edge_index[1]
    o = _matmul_bias(x, p['x2h_W'], p['x2h_b'])
    e = _matmul_bias(edge_attr, p['e2h_W'], p['e2h_b'])
    for i in range(_N_LAYERS):
        q = o @ p['tc_Wq'][i] + p['tc_bq'][i]
        k = o @ p['tc_Wk'][i] + p['tc_bk'][i]
        v = o @ p['tc_Wv'][i] + p['tc_bv'][i]
        ee = e @ p['tc_We'][i] + p['tc_be'][i]
        kj = k[src] + ee
        alpha = jnp.sum(q[dst] * kj, axis=-1) / np.sqrt(_H)
        a2 = _seg_softmax(alpha, dst, n)
        msg = (v[src] + ee) * a2[:, None]
        agg = jax.ops.segment_sum(msg, dst, num_segments=n)
        o = agg + (o @ p['tc_Ws'][i] + p['tc_bs'][i])
        m = jax.nn.relu(o[src] + e) + 1e-7
        agg2 = jax.ops.segment_sum(m, dst, num_segments=n)
        o = (agg2 + o) @ p['gen_W'][i] + p['gen_b'][i]
    sums = jax.ops.segment_sum(o, batch, num_segments=g)
    cnts = jax.ops.segment_sum(jnp.ones((n,), jnp.float32), batch, num_segments=g)
    glob = sums / jnp.maximum(cnts, 1.0)[:, None]

    def head(h, nm):
        h1 = jax.nn.leaky_relu(h @ p[nm + '_W1'] + p[nm + '_b1'], negative_slope=0.01)
        return h1 @ p[nm + '_W2'] + p[nm + '_b2']

    ne_row, ne_col = non_edge_index[0], non_edge_index[1]
    e_row, e_col = edge_index[0, ::2], edge_index[1, ::2]
    stop_logits = head(glob, 'stop')
    add_node_logits = head(o, 'add_node')
    add_edge_logits = head(o[ne_row] + o[ne_col], 'add_edge')
    add_edge_attr_logits = head(o[e_row] + o[e_col], 'add_edge_attr')
    reward = head(glob, 'reward')
    return (stop_logits, add_node_logits, add_edge_logits, add_edge_attr_logits, reward)


# speedup vs baseline: 2.5486x; 2.5486x over previous
import functools

import jax
import jax.numpy as jnp
import numpy as np
from jax import lax
from jax.experimental import pallas as pl
from jax.experimental.pallas import tpu as pltpu
from jax.experimental.pallas import tpu_sc as plsc

_NC = 2
_NS = 16
_NW = _NC * _NS
_LANES = 16
_C = 1000
_NP = 51200
_STR = _NP // _NS
_NEG = -3.0e38


def _mesh():
    return plsc.VectorSubcoreMesh(core_axis_name="c", subcore_axis_name="s")


@functools.lru_cache(maxsize=None)
def _sc_gather(tn, tw, m):
    nch = m // _C
    per = -(-nch // _NW)

    @functools.partial(
        pl.kernel,
        mesh=_mesh(),
        compiler_params=pltpu.CompilerParams(use_tc_tiling_on_sc=False),
        out_type=jax.ShapeDtypeStruct((m, tw), jnp.float32),
        scratch_types=[
            pltpu.VMEM((_C,), jnp.int32),
            pltpu.VMEM((_C, tw), jnp.float32),
            pltpu.SemaphoreType.DMA,
        ],
    )
    def gk(tab, idx, out, idx_v, rows_v, sem):
        c = lax.axis_index("c")
        s = lax.axis_index("s")
        wid = s * _NC + c

        def body(i, carry):
            ch = i * _NW + wid

            @pl.when(ch < nch)
            def _():
                base = ch * _C
                pltpu.sync_copy(idx.at[pl.ds(base, _C)], idx_v)
                pltpu.async_copy(tab.at[idx_v], rows_v, sem).wait()
                pltpu.sync_copy(rows_v, out.at[pl.ds(base, _C)])

            return carry

        lax.fori_loop(0, per, body, 0)

    return gk


def _scatter_max(mp, d, a):
    cur = plsc.load_gather(mp, [d])
    new = jnp.maximum(cur, a)
    plsc.store_scatter(mp, [d], new)
    cnt, _ = plsc.scan_count(d)

    @pl.when(jnp.max(cnt) > 1)
    def _():
        def rb(r, carry):
            chk = plsc.load_gather(mp, [d])
            plsc.store_scatter(mp, [d], jnp.maximum(chk, new), mask=chk < new)
            return carry

        lax.fori_loop(0, _LANES - 1, rb, 0)


@functools.lru_cache(maxsize=None)
def _sc_segmax(m):
    nch = m // _C
    per = -(-nch // _NW)
    nfull = _C // _LANES
    tail = _C - nfull * _LANES

    @functools.partial(
        pl.kernel,
        mesh=_mesh(),
        compiler_params=pltpu.CompilerParams(
            use_tc_tiling_on_sc=False, needs_layout_passes=False
        ),
        out_type=jax.ShapeDtypeStruct((_NC, _NP), jnp.float32),
        scratch_types=[
            pltpu.VMEM((_NP,), jnp.float32),
            pltpu.VMEM((_C + _LANES,), jnp.float32),
            pltpu.VMEM((_C + _LANES,), jnp.int32),
            pltpu.VMEM_SHARED((_NS, _NP), jnp.float32),
        ],
    )
    def kk(alpha, dst, out, mp, av, dv, shared):
        c = lax.axis_index("c")
        s = lax.axis_index("s")
        wid = s * _NC + c
        neg = jnp.full((_LANES,), _NEG, jnp.float32)

        def init(i, carry):
            mp[pl.ds(i * _LANES, _LANES)] = neg
            return carry

        lax.fori_loop(0, _NP // _LANES, init, 0)

        def chunk(i, carry):
            ch = i * _NW + wid

            @pl.when(ch < nch)
            def _():
                base = ch * _C
                pltpu.sync_copy(alpha.at[pl.ds(base, _C)], av.at[pl.ds(0, _C)])
                pltpu.sync_copy(dst.at[pl.ds(base, _C)], dv.at[pl.ds(0, _C)])

                def vec(k, carry2):
                    d = dv[pl.ds(k * _LANES, _LANES)]
                    a = av[pl.ds(k * _LANES, _LANES)]
                    _scatter_max(mp, d, a)
                    return carry2

                lax.fori_loop(0, nfull, vec, 0)
                if tail:
                    lane = lax.iota(jnp.int32, _LANES)
                    valid = lane < tail
                    d = dv[pl.ds(nfull * _LANES, _LANES)]
                    a = av[pl.ds(nfull * _LANES, _LANES)]
                    d = jnp.where(valid, d, 0)
                    a = jnp.where(valid, a, _NEG)
                    _scatter_max(mp, d, a)

            return carry

        lax.fori_loop(0, per, chunk, 0)

        pltpu.sync_copy(mp, shared.at[s])
        plsc.subcore_barrier()
        for r in range(_NS):
            pltpu.sync_copy(
                shared.at[r, pl.ds(s * _STR, _STR)], mp.at[pl.ds(r * _STR, _STR)]
            )

        def red(j, carry):
            acc = mp[pl.ds(j * _LANES, _LANES)]
            for r in range(1, _NS):
                acc = jnp.maximum(acc, mp[pl.ds(r * _STR + j * _LANES, _LANES)])
            mp[pl.ds(j * _LANES, _LANES)] = acc
            return carry

        lax.fori_loop(0, _STR // _LANES, red, 0)
        pltpu.sync_copy(mp.at[pl.ds(0, _STR)], out.at[c, pl.ds(s * _STR, _STR)])

    return kk


@functools.lru_cache(maxsize=None)
def _sc_scatter(m, with_scalar):
    nch = m // _C
    per = -(-nch // _NW)
    per2 = -(-nch // _NS)

    outs = [jax.ShapeDtypeStruct((4, _NP, 16), jnp.float32)]
    if with_scalar:
        outs.append(jax.ShapeDtypeStruct((_NP, 1), jnp.float32))

    scratch = [
        pltpu.VMEM((_C, 16), jnp.float32),
        pltpu.VMEM((_C,), jnp.int32),
        pltpu.VMEM((_C, 1), jnp.float32),
        pltpu.VMEM((_C,), jnp.int32),
        pltpu.VMEM_SHARED((_NP, 16), jnp.float32),
        pltpu.VMEM_SHARED((_NP, 1), jnp.float32),
    ]

    def body(rows4, dstr, exr, z16, z1, uout, sout, rv, dv, ev, dv2, u_sp, s_sp):
        c = lax.axis_index("c")
        s = lax.axis_index("s")
        wid = s * _NC + c

        for h in range(2):
            pltpu.sync_copy(z16, u_sp.at[pl.ds(s * _STR, _STR)])
            if with_scalar and h == 0:

                @pl.when(c == 0)
                def _():
                    pltpu.sync_copy(z1, s_sp.at[pl.ds(s * _STR, _STR)])

            plsc.subcore_barrier()

            def chunk(i, carry):
                ch = i * _NW + wid

                @pl.when(ch < nch)
                def _():
                    base = ch * _C
                    pltpu.sync_copy(rows4.at[2 * c + h, pl.ds(base, _C)], rv)
                    pltpu.sync_copy(dstr.at[pl.ds(base, _C)], dv)
                    pltpu.sync_copy(rv, u_sp.at[dv], add=True)

                return carry

            lax.fori_loop(0, per, chunk, 0)

            if with_scalar and h == 0:

                def chunk2(j, carry):
                    ch = j * _NS + s

                    @pl.when((c == 0) & (ch < nch))
                    def _():
                        base = ch * _C
                        pltpu.sync_copy(exr.at[pl.ds(base, _C)], ev)
                        pltpu.sync_copy(dstr.at[pl.ds(base, _C)], dv2)
                        pltpu.sync_copy(ev, s_sp.at[dv2], add=True)

                    return carry

                lax.fori_loop(0, per2, chunk2, 0)

            plsc.subcore_barrier()
            pltpu.sync_copy(
                u_sp.at[pl.ds(s * _STR, _STR)],
                uout.at[2 * c + h, pl.ds(s * _STR, _STR)],
            )
            if with_scalar and h == 0:

                @pl.when(c == 0)
                def _():
                    pltpu.sync_copy(s_sp.at[pl.ds(s * _STR, _STR)], sout.at[pl.ds(s * _STR, _STR)])

    if with_scalar:

        def body_ws(rows3, dstr, exr, z32, z1, uout, sout, rv, dv, ev, dv2, u_sp, s_sp):
            body(rows3, dstr, exr, z32, z1, uout, sout, rv, dv, ev, dv2, u_sp, s_sp)

        fn = body_ws
    else:

        def body_ns(rows3, dstr, z32, z1, uout, rv, dv, ev, dv2, u_sp, s_sp):
            body(rows3, dstr, None, z32, z1, uout, None, rv, dv, ev, dv2, u_sp, s_sp)

        fn = body_ns

    return functools.partial(
        pl.kernel, mesh=_mesh(),
        compiler_params=pltpu.CompilerParams(use_tc_tiling_on_sc=False),
        out_type=tuple(outs) if with_scalar else outs[0],
        scratch_types=scratch,
    )(fn)


def _scatter_zeros():
    z16 = jnp.zeros((_STR, 16), jnp.float32)
    z1 = jnp.zeros((_STR, 1), jnp.float32)
    return z16, z1


_BM = 2000


def _mm_kernel(x_ref, w_ref, b_ref, o_ref):
    o_ref[...] = (
        jnp.dot(x_ref[...], w_ref[...], preferred_element_type=jnp.float32)
        + b_ref[...]
    )


def _matmul_bias(x, w, b, bm=_BM):
    m, k = x.shape
    _, n = w.shape
    return pl.pallas_call(
        _mm_kernel,
        grid=(m // bm,),
        in_specs=[
            pl.BlockSpec((bm, k), lambda i: (i, 0)),
            pl.BlockSpec((k, n), lambda i: (0, 0)),
            pl.BlockSpec((1, n), lambda i: (0, 0)),
        ],
        out_specs=pl.BlockSpec((bm, n), lambda i: (i, 0)),
        out_shape=jax.ShapeDtypeStruct((m, n), jnp.float32),
    )(x, w, b.reshape(1, n))


def _qkvs_kernel(o_ref, wq, wk, wv, ws, bq, bk, bv, bs, q_ref, k_ref, v_ref, s_ref):
    xx = o_ref[...]
    q_ref[...] = jnp.dot(xx, wq[...], preferred_element_type=jnp.float32) + bq[...]
    k_ref[...] = jnp.dot(xx, wk[...], preferred_element_type=jnp.float32) + bk[...]
    v_ref[...] = jnp.dot(xx, wv[...], preferred_element_type=jnp.float32) + bv[...]
    s_ref[...] = jnp.dot(xx, ws[...], preferred_element_type=jnp.float32) + bs[...]


def _qkvs(o, wq, wk, wv, ws, bq, bk, bv, bs):
    n = o.shape[0]
    wspec = pl.BlockSpec((64, 64), lambda i: (0, 0))
    bspec = pl.BlockSpec((1, 64), lambda i: (0, 0))
    ospec = pl.BlockSpec((_BM, 64), lambda i: (i, 0))
    return pl.pallas_call(
        _qkvs_kernel,
        grid=(n // _BM,),
        in_specs=[ospec, wspec, wspec, wspec, wspec, bspec, bspec, bspec, bspec],
        out_specs=[ospec, ospec, ospec, ospec],
        out_shape=[jax.ShapeDtypeStruct((n, 64), jnp.float32)] * 4,
    )(o, wq, wk, wv, ws, bq.reshape(1, 64), bk.reshape(1, 64), bv.reshape(1, 64),
      bs.reshape(1, 64))


def _alpha_kernel(qg_ref, kg_ref, ee_ref, a_ref):
    prod = qg_ref[...] * (kg_ref[...] + ee_ref[...])
    a_ref[...] = jnp.sum(prod, axis=1, keepdims=True) * np.float32(0.125)


def _alpha(qg, kg, ee):
    m = qg.shape[0]
    spec = pl.BlockSpec((_BM, 64), lambda i: (i, 0))
    return pl.pallas_call(
        _alpha_kernel,
        grid=(m // _BM,),
        in_specs=[spec, spec, spec],
        out_specs=pl.BlockSpec((_BM, 1), lambda i: (i, 0)),
        out_shape=jax.ShapeDtypeStruct((m, 1), jnp.float32),
    )(qg, kg, ee)


def _maxmerge_kernel(m2_ref, o_ref):
    o_ref[...] = jnp.maximum(m2_ref[0], m2_ref[1]).reshape(_NP, 1)


def _maxmerge(m2):
    return pl.pallas_call(
        _maxmerge_kernel,
        grid=(1,),
        in_specs=[pl.BlockSpec((_NC, _NP), lambda i: (0, 0))],
        out_specs=pl.BlockSpec((_NP, 1), lambda i: (0, 0)),
        out_shape=jax.ShapeDtypeStruct((_NP, 1), jnp.float32),
    )(m2)


def _msgex_kernel(vg_ref, ee_ref, a_ref, mg_ref, pex_ref, ex_ref):
    ex = jnp.exp(a_ref[...] - mg_ref[...])
    pex = (vg_ref[...] + ee_ref[...]) * ex
    for q in range(4):
        pex_ref[q] = pex[:, 16 * q:16 * (q + 1)]
    ex_ref[...] = ex


def _msgex(vg, ee, alpha, mg):
    m = vg.shape[0]
    spec = pl.BlockSpec((_BM, 64), lambda i: (i, 0))
    spec1 = pl.BlockSpec((_BM, 1), lambda i: (i, 0))
    return pl.pallas_call(
        _msgex_kernel,
        grid=(m // _BM,),
        in_specs=[spec, spec, spec1, spec1],
        out_specs=[
            pl.BlockSpec((4, _BM, 16), lambda i: (0, i, 0)),
            spec1,
        ],
        out_shape=[
            jax.ShapeDtypeStruct((4, m, 16), jnp.float32),
            jax.ShapeDtypeStruct((m, 1), jnp.float32),
        ],
    )(vg, ee, alpha, mg)


def _combine_kernel(u_ref, s_ref, os_ref, o_ref):
    u = jnp.concatenate([u_ref[q] for q in range(4)], axis=1)
    o_ref[...] = u / (s_ref[...] + 1e-16) + os_ref[...]


def _combine(u4, sv, os_):
    n = os_.shape[0]
    spec1 = pl.BlockSpec((_BM, 1), lambda i: (i, 0))
    spec64 = pl.BlockSpec((_BM, 64), lambda i: (i, 0))
    return pl.pallas_call(
        _combine_kernel,
        grid=(n // _BM,),
        in_specs=[pl.BlockSpec((4, _BM, 16), lambda i: (0, i, 0)), spec1, spec64],
        out_specs=spec64,
        out_shape=jax.ShapeDtypeStruct((n, 64), jnp.float32),
    )(u4, sv, os_)


def _genm_kernel(og_ref, e_ref, m_ref):
    mm = jnp.maximum(og_ref[...] + e_ref[...], 0.0) + 1e-7
    for q in range(4):
        m_ref[q] = mm[:, 16 * q:16 * (q + 1)]


def _genm(og, e):
    m = og.shape[0]
    spec = pl.BlockSpec((_BM, 64), lambda i: (i, 0))
    return pl.pallas_call(
        _genm_kernel,
        grid=(m // _BM,),
        in_specs=[spec, spec],
        out_specs=pl.BlockSpec((4, _BM, 16), lambda i: (0, i, 0)),
        out_shape=jax.ShapeDtypeStruct((4, m, 16), jnp.float32),
    )(og, e)


def _genout_kernel(a_ref, o_ref, w_ref, b_ref, out_ref):
    agg = jnp.concatenate([a_ref[q] for q in range(4)], axis=1)
    t = agg + o_ref[...]
    out_ref[...] = (
        jnp.dot(t, w_ref[...], preferred_element_type=jnp.float32) + b_ref[...]
    )


def _genout(a4, o, w, b):
    n = o.shape[0]
    spec64 = pl.BlockSpec((_BM, 64), lambda i: (i, 0))
    return pl.pallas_call(
        _genout_kernel,
        grid=(n // _BM,),
        in_specs=[
            pl.BlockSpec((4, _BM, 16), lambda i: (0, i, 0)),
            spec64,
            pl.BlockSpec((64, 64), lambda i: (0, 0)),
            pl.BlockSpec((1, 64), lambda i: (0, 0)),
        ],
        out_specs=spec64,
        out_shape=jax.ShapeDtypeStruct((n, 64), jnp.float32),
    )(a4, o, w, b.reshape(1, 64))


def _glob_kernel(o_ref, b_ref, sums_ref, cnt_ref):
    @pl.when(pl.program_id(0) == 0)
    def _():
        sums_ref[...] = jnp.zeros_like(sums_ref)
        cnt_ref[...] = jnp.zeros_like(cnt_ref)

    iot = lax.broadcasted_iota(jnp.int32, (128, _BM), 0)
    oh = (iot == b_ref[0]).astype(jnp.float32)
    sums_ref[...] += jnp.dot(oh, o_ref[...], preferred_element_type=jnp.float32)
    cnt_ref[...] += jnp.sum(oh, axis=1, keepdims=True)


def _glob(o, batch2d):
    n = o.shape[0]
    return pl.pallas_call(
        _glob_kernel,
        grid=(n // _BM,),
        in_specs=[
            pl.BlockSpec((_BM, 64), lambda i: (i, 0)),
            pl.BlockSpec((1, 1, _BM), lambda i: (i, 0, 0)),
        ],
        out_specs=[
            pl.BlockSpec((128, 64), lambda i: (0, 0)),
            pl.BlockSpec((128, 1), lambda i: (0, 0)),
        ],
        out_shape=[
            jax.ShapeDtypeStruct((128, 64), jnp.float32),
            jax.ShapeDtypeStruct((128, 1), jnp.float32),
        ],
    )(o, batch2d)


def _leaky(h):
    return jnp.where(h > 0, h, 0.01 * h)


def _headg_kernel(sums_ref, cnt_ref, w1s, b1s, w2s, b2s, w1r, b1r, w2r, b2r,
                  stop_ref, rew_ref):
    glob = sums_ref[...] / jnp.maximum(cnt_ref[...], 1.0)
    h1 = _leaky(jnp.dot(glob, w1s[...], preferred_element_type=jnp.float32) + b1s[...])
    stop_ref[...] = jnp.dot(h1, w2s[...], preferred_element_type=jnp.float32) + b2s[...]
    h2 = _leaky(jnp.dot(glob, w1r[...], preferred_element_type=jnp.float32) + b1r[...])
    rew_ref[...] = jnp.dot(h2, w2r[...], preferred_element_type=jnp.float32) + b2r[...]


def _headg(sums, cnt, p):
    full = lambda shp: pl.BlockSpec(shp, lambda i: tuple(0 for _ in shp))
    return pl.pallas_call(
        _headg_kernel,
        grid=(1,),
        in_specs=[full((128, 64)), full((128, 1)),
                  full((64, 64)), full((1, 64)), full((64, 1)), full((1, 1)),
                  full((64, 64)), full((1, 64)), full((64, 1)), full((1, 1))],
        out_specs=[full((128, 1)), full((128, 1))],
        out_shape=[jax.ShapeDtypeStruct((128, 1), jnp.float32)] * 2,
    )(sums, cnt,
      p['stop_W1'], p['stop_b1'].reshape(1, 64), p['stop_W2'], p['stop_b2'].reshape(1, 1),
      p['reward_W1'], p['reward_b1'].reshape(1, 64), p['reward_W2'], p['reward_b2'].reshape(1, 1))


def _head_kernel(a_ref, w1, b1, w2, b2, o_ref):
    h1 = _leaky(jnp.dot(a_ref[...], w1[...], preferred_element_type=jnp.float32) + b1[...])
    o_ref[...] = jnp.dot(h1, w2[...], preferred_element_type=jnp.float32) + b2[...]


def _head_pair_kernel(a_ref, b_ref, w1, b1, w2, b2, o_ref):
    h = a_ref[...] + b_ref[...]
    h1 = _leaky(jnp.dot(h, w1[...], preferred_element_type=jnp.float32) + b1[...])
    o_ref[...] = jnp.dot(h1, w2[...], preferred_element_type=jnp.float32) + b2[...]


def _head(a, w1, b1, w2, b2, b=None):
    m = a.shape[0]
    nl = w2.shape[1]
    spec = pl.BlockSpec((_BM, 64), lambda i: (i, 0))
    wspecs = [
        pl.BlockSpec((64, 64), lambda i: (0, 0)),
        pl.BlockSpec((1, 64), lambda i: (0, 0)),
        pl.BlockSpec((64, nl), lambda i: (0, 0)),
        pl.BlockSpec((1, nl), lambda i: (0, 0)),
    ]
    args = [a] if b is None else [a, b]
    return pl.pallas_call(
        _head_kernel if b is None else _head_pair_kernel,
        grid=(m // _BM,),
        in_specs=[spec] * len(args) + wspecs,
        out_specs=pl.BlockSpec((_BM, nl), lambda i: (i, 0)),
        out_shape=jax.ShapeDtypeStruct((m, nl), jnp.float32),
    )(*args, w1, b1.reshape(1, 64), w2, b2.reshape(1, nl))


def kernel(x, edge_index, edge_attr, batch, non_edge_index, params):
    p = params
    n = x.shape[0]
    e_cnt = edge_index.shape[1]
    src = edge_index[0]
    dst = edge_index[1]
    z16, z1 = _scatter_zeros()

    gather_n = _sc_gather(n, 64, e_cnt)
    gather_m = _sc_gather(_NP, 1, e_cnt)
    segmax = _sc_segmax(e_cnt)
    scat_ex = _sc_scatter(e_cnt, True)
    scat_nx = _sc_scatter(e_cnt, False)

    o = _matmul_bias(x, p['x2h_W'], p['x2h_b'])
    e = _matmul_bias(edge_attr, p['e2h_W'], p['e2h_b'])

    for i in range(6):
        q, k, v, os_ = _qkvs(
            o, p['tc_Wq'][i], p['tc_Wk'][i], p['tc_Wv'][i], p['tc_Ws'][i],
            p['tc_bq'][i], p['tc_bk'][i], p['tc_bv'][i], p['tc_bs'][i])
        ee = _matmul_bias(e, p['tc_We'][i], p['tc_be'][i])
        kg = gather_n(k, src)
        qg = gather_n(q, dst)
        vg = gather_n(v, src)
        alpha = _alpha(qg, kg, ee)
        m2 = segmax(alpha.reshape(e_cnt), dst)
        mtab = _maxmerge(m2)
        mg = gather_m(mtab, dst)
        pex4, exv = _msgex(vg, ee, alpha, mg)
        u4, sv = scat_ex(pex4, dst, exv, z16, z1)
        o = _combine(u4[:, :n], sv[:n], os_)
        og = gather_n(o, src)
        m4 = _genm(og, e)
        a4 = scat_nx(m4, dst, z16, z1)
        o = _genout(a4[:, :n], o, p['gen_W'][i], p['gen_b'][i])

    sums, cnt = _glob(o, batch.reshape(n // _BM, 1, _BM))
    stop_logits, reward = _headg(sums, cnt, p)
    add_node_logits = _head(o, p['add_node_W1'], p['add_node_b1'],
                            p['add_node_W2'], p['add_node_b2'])

    ne_cnt = non_edge_index.shape[1]
    gather_ne = _sc_gather(n, 64, ne_cnt)
    oa = gather_ne(o, non_edge_index[0])
    ob = gather_ne(o, non_edge_index[1])
    add_edge_logits = _head(oa, p['add_edge_W1'], p['add_edge_b1'],
                            p['add_edge_W2'], p['add_edge_b2'], b=ob)

    er = edge_index[0, ::2]
    ec = edge_index[1, ::2]
    oc = gather_ne(o, er)
    od = gather_ne(o, ec)
    add_edge_attr_logits = _head(oc, p['add_edge_attr_W1'], p['add_edge_attr_b1'],
                                 p['add_edge_attr_W2'], p['add_edge_attr_b2'], b=od)

    return (stop_logits, add_node_logits, add_edge_logits, add_edge_attr_logits,
            reward)


# speedup vs baseline: 2.8133x; 1.1039x over previous
import functools

import jax
import jax.numpy as jnp
import numpy as np
from jax import lax
from jax.experimental import pallas as pl
from jax.experimental.pallas import tpu as pltpu
from jax.experimental.pallas import tpu_sc as plsc

_NC = 2
_NS = 16
_NW = _NC * _NS
_LANES = 16
_C = 1000
_NP = 51200
_STR = _NP // _NS
_NEG = -3.0e38


def _mesh():
    return plsc.VectorSubcoreMesh(core_axis_name="c", subcore_axis_name="s")


@functools.lru_cache(maxsize=None)
def _sc_gather(tn, tw, m):
    nch = m // _C
    per = -(-nch // _NW)

    @functools.partial(
        pl.kernel,
        mesh=_mesh(),
        compiler_params=pltpu.CompilerParams(use_tc_tiling_on_sc=False),
        out_type=jax.ShapeDtypeStruct((m, tw), jnp.float32),
        scratch_types=[
            pltpu.VMEM((_C,), jnp.int32),
            pltpu.VMEM((_C, tw), jnp.float32),
            pltpu.SemaphoreType.DMA,
        ],
    )
    def gk(tab, idx, out, idx_v, rows_v, sem):
        c = lax.axis_index("c")
        s = lax.axis_index("s")
        wid = s * _NC + c

        def body(i, carry):
            ch = i * _NW + wid

            @pl.when(ch < nch)
            def _():
                base = ch * _C
                pltpu.sync_copy(idx.at[pl.ds(base, _C)], idx_v)
                pltpu.async_copy(tab.at[idx_v], rows_v, sem).wait()
                pltpu.sync_copy(rows_v, out.at[pl.ds(base, _C)])

            return carry

        lax.fori_loop(0, per, body, 0)

    return gk


@functools.lru_cache(maxsize=None)
def _sc_gather1(tn, m):
    nch = m // _C
    per = -(-nch // _NW)

    @functools.partial(
        pl.kernel,
        mesh=_mesh(),
        compiler_params=pltpu.CompilerParams(use_tc_tiling_on_sc=False),
        out_type=jax.ShapeDtypeStruct((m,), jnp.float32),
        scratch_types=[
            pltpu.VMEM((_C,), jnp.int32),
            pltpu.VMEM((_C,), jnp.float32),
            pltpu.SemaphoreType.DMA,
        ],
    )
    def gk(tab, idx, out, idx_v, rows_v, sem):
        c = lax.axis_index("c")
        s = lax.axis_index("s")
        wid = s * _NC + c

        def body(i, carry):
            ch = i * _NW + wid

            @pl.when(ch < nch)
            def _():
                base = ch * _C
                pltpu.sync_copy(idx.at[pl.ds(base, _C)], idx_v)
                pltpu.async_copy(tab.at[idx_v], rows_v, sem).wait()
                pltpu.sync_copy(rows_v, out.at[pl.ds(base, _C)])

            return carry

        lax.fori_loop(0, per, body, 0)

    return gk


def _scatter_max(mp, d, a):
    cur = plsc.load_gather(mp, [d])
    new = jnp.maximum(cur, a)
    plsc.store_scatter(mp, [d], new)
    cnt, _ = plsc.scan_count(d)

    @pl.when(jnp.max(cnt) > 1)
    def _():
        def rb(r, carry):
            chk = plsc.load_gather(mp, [d])
            plsc.store_scatter(mp, [d], jnp.maximum(chk, new), mask=chk < new)
            return carry

        lax.fori_loop(0, _LANES - 1, rb, 0)


@functools.lru_cache(maxsize=None)
def _sc_segmax(m):
    nch = m // _C
    per = -(-nch // _NW)
    nfull = _C // _LANES
    tail = _C - nfull * _LANES

    @functools.partial(
        pl.kernel,
        mesh=_mesh(),
        compiler_params=pltpu.CompilerParams(
            use_tc_tiling_on_sc=False, needs_layout_passes=False
        ),
        out_type=jax.ShapeDtypeStruct((_NC, _NP), jnp.float32),
        scratch_types=[
            pltpu.VMEM((_NP,), jnp.float32),
            pltpu.VMEM((_C + _LANES,), jnp.float32),
            pltpu.VMEM((_C + _LANES,), jnp.int32),
            pltpu.VMEM_SHARED((_NS, _NP), jnp.float32),
        ],
    )
    def kk(alpha, dst, out, mp, av, dv, shared):
        c = lax.axis_index("c")
        s = lax.axis_index("s")
        wid = s * _NC + c
        neg = jnp.full((_LANES,), _NEG, jnp.float32)

        def init(i, carry):
            mp[pl.ds(i * _LANES, _LANES)] = neg
            return carry

        lax.fori_loop(0, _NP // _LANES, init, 0)

        def chunk(i, carry):
            ch = i * _NW + wid

            @pl.when(ch < nch)
            def _():
                base = ch * _C
                pltpu.sync_copy(alpha.at[pl.ds(base, _C)], av.at[pl.ds(0, _C)])
                pltpu.sync_copy(dst.at[pl.ds(base, _C)], dv.at[pl.ds(0, _C)])

                def vec(k, carry2):
                    d = dv[pl.ds(k * _LANES, _LANES)]
                    a = av[pl.ds(k * _LANES, _LANES)]
                    _scatter_max(mp, d, a)
                    return carry2

                lax.fori_loop(0, nfull, vec, 0)
                if tail:
                    lane = lax.iota(jnp.int32, _LANES)
                    valid = lane < tail
                    d = dv[pl.ds(nfull * _LANES, _LANES)]
                    a = av[pl.ds(nfull * _LANES, _LANES)]
                    d = jnp.where(valid, d, 0)
                    a = jnp.where(valid, a, _NEG)
                    _scatter_max(mp, d, a)

            return carry

        lax.fori_loop(0, per, chunk, 0)

        pltpu.sync_copy(mp, shared.at[s])
        plsc.subcore_barrier()
        for r in range(_NS):
            pltpu.sync_copy(
                shared.at[r, pl.ds(s * _STR, _STR)], mp.at[pl.ds(r * _STR, _STR)]
            )

        def red(j, carry):
            acc = mp[pl.ds(j * _LANES, _LANES)]
            for r in range(1, _NS):
                acc = jnp.maximum(acc, mp[pl.ds(r * _STR + j * _LANES, _LANES)])
            mp[pl.ds(j * _LANES, _LANES)] = acc
            return carry

        lax.fori_loop(0, _STR // _LANES, red, 0)
        pltpu.sync_copy(mp.at[pl.ds(0, _STR)], out.at[c, pl.ds(s * _STR, _STR)])

    return kk


@functools.lru_cache(maxsize=None)
def _sc_scatter(m, with_scalar):
    nch = m // _C
    per = -(-nch // _NW)
    per2 = -(-nch // _NS)

    outs = [jax.ShapeDtypeStruct((4, _NP, 16), jnp.float32)]
    if with_scalar:
        outs.append(jax.ShapeDtypeStruct((_NP,), jnp.float32))

    scratch = [
        pltpu.VMEM((_C, 16), jnp.float32),
        pltpu.VMEM((_C,), jnp.int32),
        pltpu.VMEM((_C,), jnp.float32),
        pltpu.VMEM((_C,), jnp.int32),
        pltpu.VMEM_SHARED((_NP, 16), jnp.float32),
        pltpu.VMEM_SHARED((_NP,), jnp.float32),
    ]

    def body(rows4, dstr, exr, z16, z1, uout, sout, rv, dv, ev, dv2, u_sp, s_sp):
        c = lax.axis_index("c")
        s = lax.axis_index("s")
        wid = s * _NC + c

        for h in range(2):
            pltpu.sync_copy(z16, u_sp.at[pl.ds(s * _STR, _STR)])
            if with_scalar and h == 0:

                @pl.when(c == 0)
                def _():
                    pltpu.sync_copy(z1, s_sp.at[pl.ds(s * _STR, _STR)])

            plsc.subcore_barrier()

            def chunk(i, carry):
                ch = i * _NW + wid

                @pl.when(ch < nch)
                def _():
                    base = ch * _C
                    pltpu.sync_copy(rows4.at[2 * c + h, pl.ds(base, _C)], rv)
                    pltpu.sync_copy(dstr.at[pl.ds(base, _C)], dv)
                    pltpu.sync_copy(rv, u_sp.at[dv], add=True)

                return carry

            lax.fori_loop(0, per, chunk, 0)

            if with_scalar and h == 0:

                def chunk2(j, carry):
                    ch = j * _NS + s

                    @pl.when((c == 0) & (ch < nch))
                    def _():
                        base = ch * _C
                        pltpu.sync_copy(exr.at[pl.ds(base, _C)], ev)
                        pltpu.sync_copy(dstr.at[pl.ds(base, _C)], dv2)
                        pltpu.sync_copy(ev, s_sp.at[dv2], add=True)

                    return carry

                lax.fori_loop(0, per2, chunk2, 0)

            plsc.subcore_barrier()
            pltpu.sync_copy(
                u_sp.at[pl.ds(s * _STR, _STR)],
                uout.at[2 * c + h, pl.ds(s * _STR, _STR)],
            )
            if with_scalar and h == 0:

                @pl.when(c == 0)
                def _():
                    pltpu.sync_copy(s_sp.at[pl.ds(s * _STR, _STR)], sout.at[pl.ds(s * _STR, _STR)])

    if with_scalar:

        def body_ws(rows3, dstr, exr, z32, z1, uout, sout, rv, dv, ev, dv2, u_sp, s_sp):
            body(rows3, dstr, exr, z32, z1, uout, sout, rv, dv, ev, dv2, u_sp, s_sp)

        fn = body_ws
    else:

        def body_ns(rows3, dstr, z32, z1, uout, rv, dv, ev, dv2, u_sp, s_sp):
            body(rows3, dstr, None, z32, z1, uout, None, rv, dv, ev, dv2, u_sp, s_sp)

        fn = body_ns

    return functools.partial(
        pl.kernel, mesh=_mesh(),
        compiler_params=pltpu.CompilerParams(use_tc_tiling_on_sc=False),
        out_type=tuple(outs) if with_scalar else outs[0],
        scratch_types=scratch,
    )(fn)


def _scatter_zeros():
    z16 = jnp.zeros((_STR, 16), jnp.float32)
    z1 = jnp.zeros((_STR,), jnp.float32)
    return z16, z1


_BM = 2000
_BME = 2048


def _mm_kernel(x_ref, w_ref, b_ref, o_ref):
    o_ref[...] = (
        jnp.dot(x_ref[...], w_ref[...], preferred_element_type=jnp.float32)
        + b_ref[...]
    )


def _matmul_bias(x, w, b, bm=_BM):
    m, k = x.shape
    _, n = w.shape
    return pl.pallas_call(
        _mm_kernel,
        grid=(m // bm,),
        in_specs=[
            pl.BlockSpec((bm, k), lambda i: (i, 0)),
            pl.BlockSpec((k, n), lambda i: (0, 0)),
            pl.BlockSpec((1, n), lambda i: (0, 0)),
        ],
        out_specs=pl.BlockSpec((bm, n), lambda i: (i, 0)),
        out_shape=jax.ShapeDtypeStruct((m, n), jnp.float32),
    )(x, w, b.reshape(1, n))


def _qkvs_kernel(o_ref, wq, wk, wv, ws, bq, bk, bv, bs, q_ref, k_ref, v_ref, s_ref):
    xx = o_ref[...]
    q_ref[...] = jnp.dot(xx, wq[...], preferred_element_type=jnp.float32) + bq[...]
    k_ref[...] = jnp.dot(xx, wk[...], preferred_element_type=jnp.float32) + bk[...]
    v_ref[...] = jnp.dot(xx, wv[...], preferred_element_type=jnp.float32) + bv[...]
    s_ref[...] = jnp.dot(xx, ws[...], preferred_element_type=jnp.float32) + bs[...]


def _qkvs(o, wq, wk, wv, ws, bq, bk, bv, bs):
    n = o.shape[0]
    wspec = pl.BlockSpec((64, 64), lambda i: (0, 0))
    bspec = pl.BlockSpec((1, 64), lambda i: (0, 0))
    ospec = pl.BlockSpec((_BM, 64), lambda i: (i, 0))
    return pl.pallas_call(
        _qkvs_kernel,
        grid=(n // _BM,),
        in_specs=[ospec, wspec, wspec, wspec, wspec, bspec, bspec, bspec, bspec],
        out_specs=[ospec, ospec, ospec, ospec],
        out_shape=[jax.ShapeDtypeStruct((n, 64), jnp.float32)] * 4,
    )(o, wq, wk, wv, ws, bq.reshape(1, 64), bk.reshape(1, 64), bv.reshape(1, 64),
      bs.reshape(1, 64))


def _alpha_kernel(qg_ref, kg_ref, ee_ref, a_ref):
    prod = qg_ref[...] * (kg_ref[...] + ee_ref[...])
    a_ref[...] = jnp.sum(prod, axis=1) * np.float32(0.125)


def _alpha(qg, kg, ee):
    m = qg.shape[0]
    spec = pl.BlockSpec((_BME, 64), lambda i: (i, 0))
    return pl.pallas_call(
        _alpha_kernel,
        grid=(pl.cdiv(m, _BME),),
        in_specs=[spec, spec, spec],
        out_specs=pl.BlockSpec((_BME,), lambda i: (i,)),
        out_shape=jax.ShapeDtypeStruct((m,), jnp.float32),
    )(qg, kg, ee)


def _maxmerge_kernel(m2_ref, o_ref):
    o_ref[...] = jnp.maximum(m2_ref[0], m2_ref[1]).reshape(_NP, 1)


def _maxmerge(m2):
    return pl.pallas_call(
        _maxmerge_kernel,
        grid=(1,),
        in_specs=[pl.BlockSpec((_NC, _NP), lambda i: (0, 0))],
        out_specs=pl.BlockSpec((_NP, 1), lambda i: (0, 0)),
        out_shape=jax.ShapeDtypeStruct((_NP, 1), jnp.float32),
    )(m2)


def _msgex_kernel(vg_ref, ee_ref, a_ref, mg_ref, pex_ref, ex_ref):
    ex = jnp.exp(a_ref[...] - mg_ref[...])
    pex = (vg_ref[...] + ee_ref[...]) * ex[:, None]
    for q in range(4):
        pex_ref[q] = pex[:, 16 * q:16 * (q + 1)]
    ex_ref[...] = ex


def _msgex(vg, ee, alpha, mg):
    m = vg.shape[0]
    spec = pl.BlockSpec((_BME, 64), lambda i: (i, 0))
    spec1 = pl.BlockSpec((_BME,), lambda i: (i,))
    return pl.pallas_call(
        _msgex_kernel,
        grid=(pl.cdiv(m, _BME),),
        in_specs=[spec, spec, spec1, spec1],
        out_specs=[
            pl.BlockSpec((4, _BME, 16), lambda i: (0, i, 0)),
            spec1,
        ],
        out_shape=[
            jax.ShapeDtypeStruct((4, m, 16), jnp.float32),
            jax.ShapeDtypeStruct((m,), jnp.float32),
        ],
    )(vg, ee, alpha, mg)


def _combine_kernel(u_ref, s_ref, os_ref, o_ref):
    u = jnp.concatenate([u_ref[q] for q in range(4)], axis=1)
    o_ref[...] = u / (s_ref[...][:, None] + 1e-16) + os_ref[...]


def _combine(u4, sv, os_):
    n = os_.shape[0]
    spec64 = pl.BlockSpec((_BME, 64), lambda i: (i, 0))
    return pl.pallas_call(
        _combine_kernel,
        grid=(pl.cdiv(n, _BME),),
        in_specs=[
            pl.BlockSpec((4, _BME, 16), lambda i: (0, i, 0)),
            pl.BlockSpec((_BME,), lambda i: (i,)),
            spec64,
        ],
        out_specs=spec64,
        out_shape=jax.ShapeDtypeStruct((n, 64), jnp.float32),
    )(u4, sv, os_)


def _genm_kernel(og_ref, e_ref, m_ref):
    mm = jnp.maximum(og_ref[...] + e_ref[...], 0.0) + 1e-7
    for q in range(4):
        m_ref[q] = mm[:, 16 * q:16 * (q + 1)]


def _genm(og, e):
    m = og.shape[0]
    spec = pl.BlockSpec((_BM, 64), lambda i: (i, 0))
    return pl.pallas_call(
        _genm_kernel,
        grid=(m // _BM,),
        in_specs=[spec, spec],
        out_specs=pl.BlockSpec((4, _BM, 16), lambda i: (0, i, 0)),
        out_shape=jax.ShapeDtypeStruct((4, m, 16), jnp.float32),
    )(og, e)


def _genout_kernel(a_ref, o_ref, w_ref, b_ref, out_ref):
    agg = jnp.concatenate([a_ref[q] for q in range(4)], axis=1)
    t = agg + o_ref[...]
    out_ref[...] = (
        jnp.dot(t, w_ref[...], preferred_element_type=jnp.float32) + b_ref[...]
    )


def _genout(a4, o, w, b):
    n = o.shape[0]
    spec64 = pl.BlockSpec((_BM, 64), lambda i: (i, 0))
    return pl.pallas_call(
        _genout_kernel,
        grid=(n // _BM,),
        in_specs=[
            pl.BlockSpec((4, _BM, 16), lambda i: (0, i, 0)),
            spec64,
            pl.BlockSpec((64, 64), lambda i: (0, 0)),
            pl.BlockSpec((1, 64), lambda i: (0, 0)),
        ],
        out_specs=spec64,
        out_shape=jax.ShapeDtypeStruct((n, 64), jnp.float32),
    )(a4, o, w, b.reshape(1, 64))


def _glob_kernel(o_ref, b_ref, sums_ref, cnt_ref):
    @pl.when(pl.program_id(0) == 0)
    def _():
        sums_ref[...] = jnp.zeros_like(sums_ref)
        cnt_ref[...] = jnp.zeros_like(cnt_ref)

    iot = lax.broadcasted_iota(jnp.int32, (128, _BM), 0)
    oh = (iot == b_ref[0]).astype(jnp.float32)
    sums_ref[...] += jnp.dot(oh, o_ref[...], preferred_element_type=jnp.float32)
    cnt_ref[...] += jnp.sum(oh, axis=1, keepdims=True)


def _glob(o, batch2d):
    n = o.shape[0]
    return pl.pallas_call(
        _glob_kernel,
        grid=(n // _BM,),
        in_specs=[
            pl.BlockSpec((_BM, 64), lambda i: (i, 0)),
            pl.BlockSpec((1, 1, _BM), lambda i: (i, 0, 0)),
        ],
        out_specs=[
            pl.BlockSpec((128, 64), lambda i: (0, 0)),
            pl.BlockSpec((128, 1), lambda i: (0, 0)),
        ],
        out_shape=[
            jax.ShapeDtypeStruct((128, 64), jnp.float32),
            jax.ShapeDtypeStruct((128, 1), jnp.float32),
        ],
    )(o, batch2d)


def _leaky(h):
    return jnp.where(h > 0, h, 0.01 * h)


def _headg_kernel(sums_ref, cnt_ref, w1s, b1s, w2s, b2s, w1r, b1r, w2r, b2r,
                  stop_ref, rew_ref):
    glob = sums_ref[...] / jnp.maximum(cnt_ref[...], 1.0)
    h1 = _leaky(jnp.dot(glob, w1s[...], preferred_element_type=jnp.float32) + b1s[...])
    stop_ref[...] = jnp.dot(h1, w2s[...], preferred_element_type=jnp.float32) + b2s[...]
    h2 = _leaky(jnp.dot(glob, w1r[...], preferred_element_type=jnp.float32) + b1r[...])
    rew_ref[...] = jnp.dot(h2, w2r[...], preferred_element_type=jnp.float32) + b2r[...]


def _headg(sums, cnt, p):
    full = lambda shp: pl.BlockSpec(shp, lambda i: tuple(0 for _ in shp))
    return pl.pallas_call(
        _headg_kernel,
        grid=(1,),
        in_specs=[full((128, 64)), full((128, 1)),
                  full((64, 64)), full((1, 64)), full((64, 1)), full((1, 1)),
                  full((64, 64)), full((1, 64)), full((64, 1)), full((1, 1))],
        out_specs=[full((128, 1)), full((128, 1))],
        out_shape=[jax.ShapeDtypeStruct((128, 1), jnp.float32)] * 2,
    )(sums, cnt,
      p['stop_W1'], p['stop_b1'].reshape(1, 64), p['stop_W2'], p['stop_b2'].reshape(1, 1),
      p['reward_W1'], p['reward_b1'].reshape(1, 64), p['reward_W2'], p['reward_b2'].reshape(1, 1))


def _head_kernel(a_ref, w1, b1, w2, b2, o_ref):
    h1 = _leaky(jnp.dot(a_ref[...], w1[...], preferred_element_type=jnp.float32) + b1[...])
    o_ref[...] = jnp.dot(h1, w2[...], preferred_element_type=jnp.float32) + b2[...]


def _head_pair_kernel(a_ref, b_ref, w1, b1, w2, b2, o_ref):
    h = a_ref[...] + b_ref[...]
    h1 = _leaky(jnp.dot(h, w1[...], preferred_element_type=jnp.float32) + b1[...])
    o_ref[...] = jnp.dot(h1, w2[...], preferred_element_type=jnp.float32) + b2[...]


def _head(a, w1, b1, w2, b2, b=None):
    m = a.shape[0]
    nl = w2.shape[1]
    spec = pl.BlockSpec((_BM, 64), lambda i: (i, 0))
    wspecs = [
        pl.BlockSpec((64, 64), lambda i: (0, 0)),
        pl.BlockSpec((1, 64), lambda i: (0, 0)),
        pl.BlockSpec((64, nl), lambda i: (0, 0)),
        pl.BlockSpec((1, nl), lambda i: (0, 0)),
    ]
    args = [a] if b is None else [a, b]
    return pl.pallas_call(
        _head_kernel if b is None else _head_pair_kernel,
        grid=(m // _BM,),
        in_specs=[spec] * len(args) + wspecs,
        out_specs=pl.BlockSpec((_BM, nl), lambda i: (i, 0)),
        out_shape=jax.ShapeDtypeStruct((m, nl), jnp.float32),
    )(*args, w1, b1.reshape(1, 64), w2, b2.reshape(1, nl))


def kernel(x, edge_index, edge_attr, batch, non_edge_index, params):
    p = params
    n = x.shape[0]
    e_cnt = edge_index.shape[1]
    src = edge_index[0]
    dst = edge_index[1]
    z16, z1 = _scatter_zeros()

    gather_n = _sc_gather(n, 64, e_cnt)
    gather_m = _sc_gather(_NP, 1, e_cnt)
    segmax = _sc_segmax(e_cnt)
    scat_ex = _sc_scatter(e_cnt, True)
    scat_nx = _sc_scatter(e_cnt, False)

    o = _matmul_bias(x, p['x2h_W'], p['x2h_b'])
    e = _matmul_bias(edge_attr, p['e2h_W'], p['e2h_b'])

    for i in range(6):
        q, k, v, os_ = _qkvs(
            o, p['tc_Wq'][i], p['tc_Wk'][i], p['tc_Wv'][i], p['tc_Ws'][i],
            p['tc_bq'][i], p['tc_bk'][i], p['tc_bv'][i], p['tc_bs'][i])
        ee = _matmul_bias(e, p['tc_We'][i], p['tc_be'][i])
        kg = gather_n(k, src)
        qg = gather_n(q, dst)
        vg = gather_n(v, src)
        alpha = _alpha(qg, kg, ee)
        m2 = segmax(alpha, dst)
        mtab = _maxmerge(m2)
        mg = gather_m(mtab, dst).reshape(e_cnt)
        pex4, exv = _msgex(vg, ee, alpha, mg)
        u4, sv = scat_ex(pex4, dst, exv, z16, z1)
        o = _combine(u4[:, :n], sv[:n], os_)
        og = gather_n(o, src)
        m4 = _genm(og, e)
        a4 = scat_nx(m4, dst, z16, z1)
        o = _genout(a4[:, :n], o, p['gen_W'][i], p['gen_b'][i])

    sums, cnt = _glob(o, batch.reshape(n // _BM, 1, _BM))
    stop_logits, reward = _headg(sums, cnt, p)
    add_node_logits = _head(o, p['add_node_W1'], p['add_node_b1'],
                            p['add_node_W2'], p['add_node_b2'])

    ne_cnt = non_edge_index.shape[1]
    gather_ne = _sc_gather(n, 64, ne_cnt)
    oa = gather_ne(o, non_edge_index[0])
    ob = gather_ne(o, non_edge_index[1])
    add_edge_logits = _head(oa, p['add_edge_W1'], p['add_edge_b1'],
                            p['add_edge_W2'], p['add_edge_b2'], b=ob)

    er = edge_index[0, ::2]
    ec = edge_index[1, ::2]
    oc = gather_ne(o, er)
    od = gather_ne(o, ec)
    add_edge_attr_logits = _head(oc, p['add_edge_attr_W1'], p['add_edge_attr_b1'],
                                 p['add_edge_attr_W2'], p['add_edge_attr_b2'], b=od)

    return (stop_logits, add_node_logits, add_edge_logits, add_edge_attr_logits,
            reward)


# speedup vs baseline: 2.8134x; 1.0000x over previous
import functools

import jax
import jax.numpy as jnp
import numpy as np
from jax import lax
from jax.experimental import pallas as pl
from jax.experimental.pallas import tpu as pltpu
from jax.experimental.pallas import tpu_sc as plsc

_NC = 2
_NS = 16
_NW = _NC * _NS
_LANES = 16
_C = 1000
_NP = 51200
_STR = _NP // _NS
_NEG = -3.0e38


def _mesh():
    return plsc.VectorSubcoreMesh(core_axis_name="c", subcore_axis_name="s")


@functools.lru_cache(maxsize=None)
def _sc_gather(tn, tw, m):
    nch = m // _C
    per = -(-nch // _NW)

    @functools.partial(
        pl.kernel,
        mesh=_mesh(),
        compiler_params=pltpu.CompilerParams(use_tc_tiling_on_sc=False),
        out_type=jax.ShapeDtypeStruct((m, tw), jnp.float32),
        scratch_types=[
            pltpu.VMEM((_C,), jnp.int32),
            pltpu.VMEM((_C, tw), jnp.float32),
            pltpu.SemaphoreType.DMA,
        ],
    )
    def gk(tab, idx, out, idx_v, rows_v, sem):
        c = lax.axis_index("c")
        s = lax.axis_index("s")
        wid = s * _NC + c

        def body(i, carry):
            ch = i * _NW + wid

            @pl.when(ch < nch)
            def _():
                base = ch * _C
                pltpu.sync_copy(idx.at[pl.ds(base, _C)], idx_v)
                pltpu.async_copy(tab.at[idx_v], rows_v, sem).wait()
                pltpu.sync_copy(rows_v, out.at[pl.ds(base, _C)])

            return carry

        lax.fori_loop(0, per, body, 0)

    return gk


def _scatter_max(mp, d, a):
    cur = plsc.load_gather(mp, [d])
    new = jnp.maximum(cur, a)
    plsc.store_scatter(mp, [d], new)
    cnt, _ = plsc.scan_count(d)

    @pl.when(jnp.max(cnt) > 1)
    def _():
        def rb(r, carry):
            chk = plsc.load_gather(mp, [d])
            plsc.store_scatter(mp, [d], jnp.maximum(chk, new), mask=chk < new)
            return carry

        lax.fori_loop(0, _LANES - 1, rb, 0)


@functools.lru_cache(maxsize=None)
def _sc_segmax(m):
    nch = m // _C
    per = -(-nch // _NW)
    nfull = _C // _LANES
    tail = _C - nfull * _LANES

    @functools.partial(
        pl.kernel,
        mesh=_mesh(),
        compiler_params=pltpu.CompilerParams(
            use_tc_tiling_on_sc=False, needs_layout_passes=False
        ),
        out_type=jax.ShapeDtypeStruct((_NC, _NP), jnp.float32),
        scratch_types=[
            pltpu.VMEM((_NP,), jnp.float32),
            pltpu.VMEM((_C + _LANES,), jnp.float32),
            pltpu.VMEM((_C + _LANES,), jnp.int32),
            pltpu.VMEM_SHARED((_NS, _NP), jnp.float32),
        ],
    )
    def kk(alpha, dst, out, mp, av, dv, shared):
        c = lax.axis_index("c")
        s = lax.axis_index("s")
        wid = s * _NC + c
        neg = jnp.full((_LANES,), _NEG, jnp.float32)

        def init(i, carry):
            mp[pl.ds(i * _LANES, _LANES)] = neg
            return carry

        lax.fori_loop(0, _NP // _LANES, init, 0)

        def chunk(i, carry):
            ch = i * _NW + wid

            @pl.when(ch < nch)
            def _():
                base = ch * _C
                pltpu.sync_copy(alpha.at[pl.ds(base, _C)], av.at[pl.ds(0, _C)])
                pltpu.sync_copy(dst.at[pl.ds(base, _C)], dv.at[pl.ds(0, _C)])

                def vec(k, carry2):
                    d = dv[pl.ds(k * _LANES, _LANES)]
                    a = av[pl.ds(k * _LANES, _LANES)]
                    _scatter_max(mp, d, a)
                    return carry2

                lax.fori_loop(0, nfull, vec, 0)
                if tail:
                    lane = lax.iota(jnp.int32, _LANES)
                    valid = lane < tail
                    d = dv[pl.ds(nfull * _LANES, _LANES)]
                    a = av[pl.ds(nfull * _LANES, _LANES)]
                    d = jnp.where(valid, d, 0)
                    a = jnp.where(valid, a, _NEG)
                    _scatter_max(mp, d, a)

            return carry

        lax.fori_loop(0, per, chunk, 0)

        pltpu.sync_copy(mp, shared.at[s])
        plsc.subcore_barrier()
        for r in range(_NS):
            pltpu.sync_copy(
                shared.at[r, pl.ds(s * _STR, _STR)], mp.at[pl.ds(r * _STR, _STR)]
            )

        def red(j, carry):
            acc = mp[pl.ds(j * _LANES, _LANES)]
            for r in range(1, _NS):
                acc = jnp.maximum(acc, mp[pl.ds(r * _STR + j * _LANES, _LANES)])
            mp[pl.ds(j * _LANES, _LANES)] = acc
            return carry

        lax.fori_loop(0, _STR // _LANES, red, 0)
        pltpu.sync_copy(mp.at[pl.ds(0, _STR)], out.at[c, pl.ds(s * _STR, _STR)])

    return kk


@functools.lru_cache(maxsize=None)
def _sc_scatter(m, with_scalar):
    nch = m // _C
    per = -(-nch // _NW)
    per2 = -(-nch // _NS)

    outs = [jax.ShapeDtypeStruct((4, _NP, 16), jnp.float32)]
    if with_scalar:
        outs.append(jax.ShapeDtypeStruct((_NP,), jnp.float32))

    scratch = [
        pltpu.VMEM((_C, 16), jnp.float32),
        pltpu.VMEM((_C,), jnp.int32),
        pltpu.VMEM((_C,), jnp.float32),
        pltpu.VMEM((_C,), jnp.int32),
        pltpu.VMEM_SHARED((_NP, 16), jnp.float32),
        pltpu.VMEM_SHARED((_NP,), jnp.float32),
    ]

    def body(rows4, dstr, exr, z16, z1, uout, sout, rv, dv, ev, dv2, u_sp, s_sp):
        c = lax.axis_index("c")
        s = lax.axis_index("s")
        wid = s * _NC + c

        for h in range(2):
            pltpu.sync_copy(z16, u_sp.at[pl.ds(s * _STR, _STR)])
            if with_scalar and h == 0:

                @pl.when(c == 0)
                def _():
                    pltpu.sync_copy(z1, s_sp.at[pl.ds(s * _STR, _STR)])

            plsc.subcore_barrier()

            def chunk(i, carry):
                ch = i * _NW + wid

                @pl.when(ch < nch)
                def _():
                    base = ch * _C
                    pltpu.sync_copy(rows4.at[2 * c + h, pl.ds(base, _C)], rv)
                    pltpu.sync_copy(dstr.at[pl.ds(base, _C)], dv)
                    pltpu.sync_copy(rv, u_sp.at[dv], add=True)

                return carry

            lax.fori_loop(0, per, chunk, 0)

            if with_scalar and h == 0:

                def chunk2(j, carry):
                    ch = j * _NS + s

                    @pl.when((c == 0) & (ch < nch))
                    def _():
                        base = ch * _C
                        pltpu.sync_copy(exr.at[pl.ds(base, _C)], ev)
                        pltpu.sync_copy(dstr.at[pl.ds(base, _C)], dv2)
                        pltpu.sync_copy(ev, s_sp.at[dv2], add=True)

                    return carry

                lax.fori_loop(0, per2, chunk2, 0)

            plsc.subcore_barrier()
            pltpu.sync_copy(
                u_sp.at[pl.ds(s * _STR, _STR)],
                uout.at[2 * c + h, pl.ds(s * _STR, _STR)],
            )
            if with_scalar and h == 0:

                @pl.when(c == 0)
                def _():
                    pltpu.sync_copy(s_sp.at[pl.ds(s * _STR, _STR)], sout.at[pl.ds(s * _STR, _STR)])

    if with_scalar:

        def body_ws(rows3, dstr, exr, z32, z1, uout, sout, rv, dv, ev, dv2, u_sp, s_sp):
            body(rows3, dstr, exr, z32, z1, uout, sout, rv, dv, ev, dv2, u_sp, s_sp)

        fn = body_ws
    else:

        def body_ns(rows3, dstr, z32, z1, uout, rv, dv, ev, dv2, u_sp, s_sp):
            body(rows3, dstr, None, z32, z1, uout, None, rv, dv, ev, dv2, u_sp, s_sp)

        fn = body_ns

    return functools.partial(
        pl.kernel, mesh=_mesh(),
        compiler_params=pltpu.CompilerParams(use_tc_tiling_on_sc=False),
        out_type=tuple(outs) if with_scalar else outs[0],
        scratch_types=scratch,
    )(fn)


def _scatter_zeros():
    z16 = jnp.zeros((_STR, 16), jnp.float32)
    z1 = jnp.zeros((_STR,), jnp.float32)
    return z16, z1


_BM = 2000
_BME = 2048


def _mm_kernel(x_ref, w_ref, b_ref, o_ref):
    o_ref[...] = (
        jnp.dot(x_ref[...], w_ref[...], preferred_element_type=jnp.float32)
        + b_ref[...]
    )


def _matmul_bias(x, w, b, bm=_BM):
    m, k = x.shape
    _, n = w.shape
    return pl.pallas_call(
        _mm_kernel,
        grid=(m // bm,),
        in_specs=[
            pl.BlockSpec((bm, k), lambda i: (i, 0)),
            pl.BlockSpec((k, n), lambda i: (0, 0)),
            pl.BlockSpec((1, n), lambda i: (0, 0)),
        ],
        out_specs=pl.BlockSpec((bm, n), lambda i: (i, 0)),
        out_shape=jax.ShapeDtypeStruct((m, n), jnp.float32),
    )(x, w, b.reshape(1, n))


def _qkvs_kernel(o_ref, wq, wk, wv, ws, bq, bk, bv, bs, q_ref, k_ref, v_ref, s_ref):
    xx = o_ref[...]
    q_ref[...] = jnp.dot(xx, wq[...], preferred_element_type=jnp.float32) + bq[...]
    k_ref[...] = jnp.dot(xx, wk[...], preferred_element_type=jnp.float32) + bk[...]
    v_ref[...] = jnp.dot(xx, wv[...], preferred_element_type=jnp.float32) + bv[...]
    s_ref[...] = jnp.dot(xx, ws[...], preferred_element_type=jnp.float32) + bs[...]


def _qkvs(o, wq, wk, wv, ws, bq, bk, bv, bs):
    n = o.shape[0]
    wspec = pl.BlockSpec((64, 64), lambda i: (0, 0))
    bspec = pl.BlockSpec((1, 64), lambda i: (0, 0))
    ospec = pl.BlockSpec((_BM, 64), lambda i: (i, 0))
    return pl.pallas_call(
        _qkvs_kernel,
        grid=(n // _BM,),
        in_specs=[ospec, wspec, wspec, wspec, wspec, bspec, bspec, bspec, bspec],
        out_specs=[ospec, ospec, ospec, ospec],
        out_shape=[jax.ShapeDtypeStruct((n, 64), jnp.float32)] * 4,
    )(o, wq, wk, wv, ws, bq.reshape(1, 64), bk.reshape(1, 64), bv.reshape(1, 64),
      bs.reshape(1, 64))


def _alpha_kernel(qg_ref, kg_ref, ee_ref, a_ref):
    prod = qg_ref[...] * (kg_ref[...] + ee_ref[...])
    a_ref[...] = jnp.sum(prod, axis=1) * np.float32(0.125)


def _alpha(qg, kg, ee):
    m = qg.shape[0]
    spec = pl.BlockSpec((_BME, 64), lambda i: (i, 0))
    return pl.pallas_call(
        _alpha_kernel,
        grid=(pl.cdiv(m, _BME),),
        in_specs=[spec, spec, spec],
        out_specs=pl.BlockSpec((_BME,), lambda i: (i,)),
        out_shape=jax.ShapeDtypeStruct((m,), jnp.float32),
    )(qg, kg, ee)


def _maxmerge_kernel(m2_ref, o_ref):
    o_ref[...] = jnp.maximum(m2_ref[0], m2_ref[1]).reshape(_NP, 1)


def _maxmerge(m2):
    return pl.pallas_call(
        _maxmerge_kernel,
        grid=(1,),
        in_specs=[pl.BlockSpec((_NC, _NP), lambda i: (0, 0))],
        out_specs=pl.BlockSpec((_NP, 1), lambda i: (0, 0)),
        out_shape=jax.ShapeDtypeStruct((_NP, 1), jnp.float32),
    )(m2)


def _msgex_kernel(vg_ref, ee_ref, a_ref, mg_ref, pex_ref, ex_ref):
    ex = jnp.exp(a_ref[...] - mg_ref[...])
    pex = (vg_ref[...] + ee_ref[...]) * ex[:, None]
    for q in range(4):
        pex_ref[q] = pex[:, 16 * q:16 * (q + 1)]
    ex_ref[...] = ex


def _msgex(vg, ee, alpha, mg):
    m = vg.shape[0]
    spec = pl.BlockSpec((_BME, 64), lambda i: (i, 0))
    spec1 = pl.BlockSpec((_BME,), lambda i: (i,))
    return pl.pallas_call(
        _msgex_kernel,
        grid=(pl.cdiv(m, _BME),),
        in_specs=[spec, spec, spec1, spec1],
        out_specs=[
            pl.BlockSpec((4, _BME, 16), lambda i: (0, i, 0)),
            spec1,
        ],
        out_shape=[
            jax.ShapeDtypeStruct((4, m, 16), jnp.float32),
            jax.ShapeDtypeStruct((m,), jnp.float32),
        ],
    )(vg, ee, alpha, mg)


def _combine_kernel(u_ref, s_ref, os_ref, o_ref):
    u = jnp.concatenate([u_ref[q] for q in range(4)], axis=1)
    o_ref[...] = u / (s_ref[...][:, None] + 1e-16) + os_ref[...]


def _combine(u4, sv, os_):
    n = os_.shape[0]
    spec64 = pl.BlockSpec((_BME, 64), lambda i: (i, 0))
    return pl.pallas_call(
        _combine_kernel,
        grid=(pl.cdiv(n, _BME),),
        in_specs=[
            pl.BlockSpec((4, _BME, 16), lambda i: (0, i, 0)),
            pl.BlockSpec((_BME,), lambda i: (i,)),
            spec64,
        ],
        out_specs=spec64,
        out_shape=jax.ShapeDtypeStruct((n, 64), jnp.float32),
    )(u4, sv, os_)


def _genm_kernel(og_ref, e_ref, m_ref):
    mm = jnp.maximum(og_ref[...] + e_ref[...], 0.0) + 1e-7
    for q in range(4):
        m_ref[q] = mm[:, 16 * q:16 * (q + 1)]


def _genm(og, e):
    m = og.shape[0]
    spec = pl.BlockSpec((_BM, 64), lambda i: (i, 0))
    return pl.pallas_call(
        _genm_kernel,
        grid=(m // _BM,),
        in_specs=[spec, spec],
        out_specs=pl.BlockSpec((4, _BM, 16), lambda i: (0, i, 0)),
        out_shape=jax.ShapeDtypeStruct((4, m, 16), jnp.float32),
    )(og, e)


def _genout_kernel(a_ref, o_ref, w_ref, b_ref, out_ref):
    agg = jnp.concatenate([a_ref[q] for q in range(4)], axis=1)
    t = agg + o_ref[...]
    out_ref[...] = (
        jnp.dot(t, w_ref[...], preferred_element_type=jnp.float32) + b_ref[...]
    )


def _genout(a4, o, w, b):
    n = o.shape[0]
    spec64 = pl.BlockSpec((_BM, 64), lambda i: (i, 0))
    return pl.pallas_call(
        _genout_kernel,
        grid=(n // _BM,),
        in_specs=[
            pl.BlockSpec((4, _BM, 16), lambda i: (0, i, 0)),
            spec64,
            pl.BlockSpec((64, 64), lambda i: (0, 0)),
            pl.BlockSpec((1, 64), lambda i: (0, 0)),
        ],
        out_specs=spec64,
        out_shape=jax.ShapeDtypeStruct((n, 64), jnp.float32),
    )(a4, o, w, b.reshape(1, 64))


def _glob_kernel(o_ref, b_ref, sums_ref, cnt_ref):
    @pl.when(pl.program_id(0) == 0)
    def _():
        sums_ref[...] = jnp.zeros_like(sums_ref)
        cnt_ref[...] = jnp.zeros_like(cnt_ref)

    iot = lax.broadcasted_iota(jnp.int32, (128, _BM), 0)
    oh = (iot == b_ref[0]).astype(jnp.float32)
    sums_ref[...] += jnp.dot(oh, o_ref[...], preferred_element_type=jnp.float32)
    cnt_ref[...] += jnp.sum(oh, axis=1, keepdims=True)


def _glob(o, batch2d):
    n = o.shape[0]
    return pl.pallas_call(
        _glob_kernel,
        grid=(n // _BM,),
        in_specs=[
            pl.BlockSpec((_BM, 64), lambda i: (i, 0)),
            pl.BlockSpec((1, 1, _BM), lambda i: (i, 0, 0)),
        ],
        out_specs=[
            pl.BlockSpec((128, 64), lambda i: (0, 0)),
            pl.BlockSpec((128, 1), lambda i: (0, 0)),
        ],
        out_shape=[
            jax.ShapeDtypeStruct((128, 64), jnp.float32),
            jax.ShapeDtypeStruct((128, 1), jnp.float32),
        ],
    )(o, batch2d)


def _leaky(h):
    return jnp.where(h > 0, h, 0.01 * h)


def _headg_kernel(sums_ref, cnt_ref, w1s, b1s, w2s, b2s, w1r, b1r, w2r, b2r,
                  stop_ref, rew_ref):
    glob = sums_ref[...] / jnp.maximum(cnt_ref[...], 1.0)
    h1 = _leaky(jnp.dot(glob, w1s[...], preferred_element_type=jnp.float32) + b1s[...])
    stop_ref[...] = jnp.dot(h1, w2s[...], preferred_element_type=jnp.float32) + b2s[...]
    h2 = _leaky(jnp.dot(glob, w1r[...], preferred_element_type=jnp.float32) + b1r[...])
    rew_ref[...] = jnp.dot(h2, w2r[...], preferred_element_type=jnp.float32) + b2r[...]


def _headg(sums, cnt, p):
    full = lambda shp: pl.BlockSpec(shp, lambda i: tuple(0 for _ in shp))
    return pl.pallas_call(
        _headg_kernel,
        grid=(1,),
        in_specs=[full((128, 64)), full((128, 1)),
                  full((64, 64)), full((1, 64)), full((64, 1)), full((1, 1)),
                  full((64, 64)), full((1, 64)), full((64, 1)), full((1, 1))],
        out_specs=[full((128, 1)), full((128, 1))],
        out_shape=[jax.ShapeDtypeStruct((128, 1), jnp.float32)] * 2,
    )(sums, cnt,
      p['stop_W1'], p['stop_b1'].reshape(1, 64), p['stop_W2'], p['stop_b2'].reshape(1, 1),
      p['reward_W1'], p['reward_b1'].reshape(1, 64), p['reward_W2'], p['reward_b2'].reshape(1, 1))


def _head_kernel(a_ref, w1, b1, w2, b2, o_ref):
    h1 = _leaky(jnp.dot(a_ref[...], w1[...], preferred_element_type=jnp.float32) + b1[...])
    o_ref[...] = jnp.dot(h1, w2[...], preferred_element_type=jnp.float32) + b2[...]


def _head_pair_kernel(a_ref, b_ref, w1, b1, w2, b2, o_ref):
    h = a_ref[...] + b_ref[...]
    h1 = _leaky(jnp.dot(h, w1[...], preferred_element_type=jnp.float32) + b1[...])
    o_ref[...] = jnp.dot(h1, w2[...], preferred_element_type=jnp.float32) + b2[...]


def _head(a, w1, b1, w2, b2, b=None):
    m = a.shape[0]
    nl = w2.shape[1]
    spec = pl.BlockSpec((_BM, 64), lambda i: (i, 0))
    wspecs = [
        pl.BlockSpec((64, 64), lambda i: (0, 0)),
        pl.BlockSpec((1, 64), lambda i: (0, 0)),
        pl.BlockSpec((64, nl), lambda i: (0, 0)),
        pl.BlockSpec((1, nl), lambda i: (0, 0)),
    ]
    args = [a] if b is None else [a, b]
    return pl.pallas_call(
        _head_kernel if b is None else _head_pair_kernel,
        grid=(m // _BM,),
        in_specs=[spec] * len(args) + wspecs,
        out_specs=pl.BlockSpec((_BM, nl), lambda i: (i, 0)),
        out_shape=jax.ShapeDtypeStruct((m, nl), jnp.float32),
    )(*args, w1, b1.reshape(1, 64), w2, b2.reshape(1, nl))


def kernel(x, edge_index, edge_attr, batch, non_edge_index, params):
    p = params
    n = x.shape[0]
    e_cnt = edge_index.shape[1]
    src = edge_index[0]
    dst = edge_index[1]
    z16, z1 = _scatter_zeros()

    gather_n = _sc_gather(n, 64, e_cnt)
    gather_m = _sc_gather(_NP, 1, e_cnt)
    segmax = _sc_segmax(e_cnt)
    scat_ex = _sc_scatter(e_cnt, True)
    scat_nx = _sc_scatter(e_cnt, False)

    o = _matmul_bias(x, p['x2h_W'], p['x2h_b'])
    e = _matmul_bias(edge_attr, p['e2h_W'], p['e2h_b'])

    for i in range(6):
        q, k, v, os_ = _qkvs(
            o, p['tc_Wq'][i], p['tc_Wk'][i], p['tc_Wv'][i], p['tc_Ws'][i],
            p['tc_bq'][i], p['tc_bk'][i], p['tc_bv'][i], p['tc_bs'][i])
        ee = _matmul_bias(e, p['tc_We'][i], p['tc_be'][i])
        kg = gather_n(k, src)
        qg = gather_n(q, dst)
        vg = gather_n(v, src)
        alpha = _alpha(qg, kg, ee)
        m2 = segmax(alpha, dst)
        mtab = _maxmerge(m2)
        mg = gather_m(mtab, dst).reshape(e_cnt)
        pex4, exv = _msgex(vg, ee, alpha, mg)
        u4, sv = scat_ex(pex4, dst, exv, z16, z1)
        o = _combine(u4[:, :n], sv[:n], os_)
        og = gather_n(o, src)
        m4 = _genm(og, e)
        a4 = scat_nx(m4, dst, z16, z1)
        o = _genout(a4[:, :n], o, p['gen_W'][i], p['gen_b'][i])

    sums, cnt = _glob(o, batch.reshape(n // _BM, 1, _BM))
    stop_logits, reward = _headg(sums, cnt, p)
    add_node_logits = _head(o, p['add_node_W1'], p['add_node_b1'],
                            p['add_node_W2'], p['add_node_b2'])

    ne_cnt = non_edge_index.shape[1]
    gather_ne = _sc_gather(n, 64, ne_cnt)
    oa = gather_ne(o, non_edge_index[0])
    ob = gather_ne(o, non_edge_index[1])
    add_edge_logits = _head(oa, p['add_edge_W1'], p['add_edge_b1'],
                            p['add_edge_W2'], p['add_edge_b2'], b=ob)

    er = edge_index[0, ::2]
    ec = edge_index[1, ::2]
    oc = gather_ne(o, er)
    od = gather_ne(o, ec)
    add_edge_attr_logits = _head(oc, p['add_edge_attr_W1'], p['add_edge_attr_b1'],
                                 p['add_edge_attr_W2'], p['add_edge_attr_b2'], b=od)

    return (stop_logits, add_node_logits, add_edge_logits, add_edge_attr_logits,
            reward)


# speedup vs baseline: 2.8247x; 1.0040x over previous
import functools

import jax
import jax.numpy as jnp
import numpy as np
from jax import lax
from jax.experimental import pallas as pl
from jax.experimental.pallas import tpu as pltpu
from jax.experimental.pallas import tpu_sc as plsc

_NC = 2
_NS = 16
_NW = _NC * _NS
_LANES = 16
_C = 1000
_NP = 51200
_STR = _NP // _NS
_NEG = -3.0e38


def _mesh():
    return plsc.VectorSubcoreMesh(core_axis_name="c", subcore_axis_name="s")


@functools.lru_cache(maxsize=None)
def _sc_gather(tn, tw, m):
    nch = m // _C
    per = -(-nch // _NW)

    @functools.partial(
        pl.kernel,
        mesh=_mesh(),
        compiler_params=pltpu.CompilerParams(use_tc_tiling_on_sc=False),
        out_type=jax.ShapeDtypeStruct((m, tw), jnp.float32),
        scratch_types=[
            pltpu.VMEM((_C,), jnp.int32),
            pltpu.VMEM((_C, tw), jnp.float32),
            pltpu.SemaphoreType.DMA,
        ],
    )
    def gk(tab, idx, out, idx_v, rows_v, sem):
        c = lax.axis_index("c")
        s = lax.axis_index("s")
        wid = s * _NC + c

        def body(i, carry):
            ch = i * _NW + wid

            @pl.when(ch < nch)
            def _():
                base = ch * _C
                pltpu.sync_copy(idx.at[pl.ds(base, _C)], idx_v)
                pltpu.async_copy(tab.at[idx_v], rows_v, sem).wait()
                pltpu.sync_copy(rows_v, out.at[pl.ds(base, _C)])

            return carry

        lax.fori_loop(0, per, body, 0)

    return gk


@functools.lru_cache(maxsize=None)
def _sc_gather_qkv(tn, m):
    nch = m // _C
    per = -(-nch // _NW)

    @functools.partial(
        pl.kernel,
        mesh=_mesh(),
        compiler_params=pltpu.CompilerParams(use_tc_tiling_on_sc=False),
        out_type=tuple(
            jax.ShapeDtypeStruct((m, 64), jnp.float32) for _ in range(3)
        ),
        scratch_types=[
            pltpu.VMEM((_C,), jnp.int32),
            pltpu.VMEM((_C, 64), jnp.float32),
            pltpu.SemaphoreType.DMA,
        ],
    )
    def gk(qt, kt, vt, srcr, dstr, qo, ko, vo, idx_v, rows_v, sem):
        c = lax.axis_index("c")
        s = lax.axis_index("s")
        wid = s * _NC + c

        def body(i, carry):
            ch = i * _NW + wid

            @pl.when(ch < nch)
            def _():
                base = ch * _C
                pltpu.sync_copy(srcr.at[pl.ds(base, _C)], idx_v)
                pltpu.async_copy(kt.at[idx_v], rows_v, sem).wait()
                pltpu.sync_copy(rows_v, ko.at[pl.ds(base, _C)])
                pltpu.async_copy(vt.at[idx_v], rows_v, sem).wait()
                pltpu.sync_copy(rows_v, vo.at[pl.ds(base, _C)])
                pltpu.sync_copy(dstr.at[pl.ds(base, _C)], idx_v)
                pltpu.async_copy(qt.at[idx_v], rows_v, sem).wait()
                pltpu.sync_copy(rows_v, qo.at[pl.ds(base, _C)])

            return carry

        lax.fori_loop(0, per, body, 0)

    return gk


def _scatter_max(mp, d, a):
    cur = plsc.load_gather(mp, [d])
    new = jnp.maximum(cur, a)
    plsc.store_scatter(mp, [d], new)
    cnt, _ = plsc.scan_count(d)

    @pl.when(jnp.max(cnt) > 1)
    def _():
        def rb(r, carry):
            chk = plsc.load_gather(mp, [d])
            plsc.store_scatter(mp, [d], jnp.maximum(chk, new), mask=chk < new)
            return carry

        lax.fori_loop(0, _LANES - 1, rb, 0)


@functools.lru_cache(maxsize=None)
def _sc_segmax(m):
    nch = m // _C
    per = -(-nch // _NW)
    nfull = _C // _LANES
    tail = _C - nfull * _LANES

    @functools.partial(
        pl.kernel,
        mesh=_mesh(),
        compiler_params=pltpu.CompilerParams(
            use_tc_tiling_on_sc=False, needs_layout_passes=False
        ),
        out_type=jax.ShapeDtypeStruct((_NC, _NP), jnp.float32),
        scratch_types=[
            pltpu.VMEM((_NP,), jnp.float32),
            pltpu.VMEM((_C + _LANES,), jnp.float32),
            pltpu.VMEM((_C + _LANES,), jnp.int32),
            pltpu.VMEM_SHARED((_NS, _NP), jnp.float32),
        ],
    )
    def kk(alpha, dst, out, mp, av, dv, shared):
        c = lax.axis_index("c")
        s = lax.axis_index("s")
        wid = s * _NC + c
        neg = jnp.full((_LANES,), _NEG, jnp.float32)

        def init(i, carry):
            mp[pl.ds(i * _LANES, _LANES)] = neg
            return carry

        lax.fori_loop(0, _NP // _LANES, init, 0)

        def chunk(i, carry):
            ch = i * _NW + wid

            @pl.when(ch < nch)
            def _():
                base = ch * _C
                pltpu.sync_copy(alpha.at[pl.ds(base, _C)], av.at[pl.ds(0, _C)])
                pltpu.sync_copy(dst.at[pl.ds(base, _C)], dv.at[pl.ds(0, _C)])

                def vec(k, carry2):
                    d = dv[pl.ds(k * _LANES, _LANES)]
                    a = av[pl.ds(k * _LANES, _LANES)]
                    _scatter_max(mp, d, a)
                    return carry2

                lax.fori_loop(0, nfull, vec, 0)
                if tail:
                    lane = lax.iota(jnp.int32, _LANES)
                    valid = lane < tail
                    d = dv[pl.ds(nfull * _LANES, _LANES)]
                    a = av[pl.ds(nfull * _LANES, _LANES)]
                    d = jnp.where(valid, d, 0)
                    a = jnp.where(valid, a, _NEG)
                    _scatter_max(mp, d, a)

            return carry

        lax.fori_loop(0, per, chunk, 0)

        pltpu.sync_copy(mp, shared.at[s])
        plsc.subcore_barrier()
        for r in range(_NS):
            pltpu.sync_copy(
                shared.at[r, pl.ds(s * _STR, _STR)], mp.at[pl.ds(r * _STR, _STR)]
            )

        def red(j, carry):
            acc = mp[pl.ds(j * _LANES, _LANES)]
            for r in range(1, _NS):
                acc = jnp.maximum(acc, mp[pl.ds(r * _STR + j * _LANES, _LANES)])
            mp[pl.ds(j * _LANES, _LANES)] = acc
            return carry

        lax.fori_loop(0, _STR // _LANES, red, 0)
        pltpu.sync_copy(mp.at[pl.ds(0, _STR)], out.at[c, pl.ds(s * _STR, _STR)])

    return kk


@functools.lru_cache(maxsize=None)
def _sc_scatter(m, with_scalar):
    nch = m // _C
    per = -(-nch // _NW)
    per2 = -(-nch // _NS)

    outs = [jax.ShapeDtypeStruct((4, _NP, 16), jnp.float32)]
    if with_scalar:
        outs.append(jax.ShapeDtypeStruct((_NP,), jnp.float32))

    scratch = [
        pltpu.VMEM((_C, 16), jnp.float32),
        pltpu.VMEM((_C,), jnp.int32),
        pltpu.VMEM((_C,), jnp.float32),
        pltpu.VMEM((_C,), jnp.int32),
        pltpu.VMEM_SHARED((_NP, 16), jnp.float32),
        pltpu.VMEM_SHARED((_NP,), jnp.float32),
    ]

    def body(rows4, dstr, exr, z16, z1, uout, sout, rv, dv, ev, dv2, u_sp, s_sp):
        c = lax.axis_index("c")
        s = lax.axis_index("s")
        wid = s * _NC + c

        for h in range(2):
            pltpu.sync_copy(z16, u_sp.at[pl.ds(s * _STR, _STR)])
            if with_scalar and h == 0:

                @pl.when(c == 0)
                def _():
                    pltpu.sync_copy(z1, s_sp.at[pl.ds(s * _STR, _STR)])

            plsc.subcore_barrier()

            def chunk(i, carry):
                ch = i * _NW + wid

                @pl.when(ch < nch)
                def _():
                    base = ch * _C
                    pltpu.sync_copy(rows4.at[2 * c + h, pl.ds(base, _C)], rv)
                    pltpu.sync_copy(dstr.at[pl.ds(base, _C)], dv)
                    pltpu.sync_copy(rv, u_sp.at[dv], add=True)

                return carry

            lax.fori_loop(0, per, chunk, 0)

            if with_scalar and h == 0:

                def chunk2(j, carry):
                    ch = j * _NS + s

                    @pl.when((c == 0) & (ch < nch))
                    def _():
                        base = ch * _C
                        pltpu.sync_copy(exr.at[pl.ds(base, _C)], ev)
                        pltpu.sync_copy(dstr.at[pl.ds(base, _C)], dv2)
                        pltpu.sync_copy(ev, s_sp.at[dv2], add=True)

                    return carry

                lax.fori_loop(0, per2, chunk2, 0)

            plsc.subcore_barrier()
            pltpu.sync_copy(
                u_sp.at[pl.ds(s * _STR, _STR)],
                uout.at[2 * c + h, pl.ds(s * _STR, _STR)],
            )
            if with_scalar and h == 0:

                @pl.when(c == 0)
                def _():
                    pltpu.sync_copy(s_sp.at[pl.ds(s * _STR, _STR)], sout.at[pl.ds(s * _STR, _STR)])

    if with_scalar:

        def body_ws(rows3, dstr, exr, z32, z1, uout, sout, rv, dv, ev, dv2, u_sp, s_sp):
            body(rows3, dstr, exr, z32, z1, uout, sout, rv, dv, ev, dv2, u_sp, s_sp)

        fn = body_ws
    else:

        def body_ns(rows3, dstr, z32, z1, uout, rv, dv, ev, dv2, u_sp, s_sp):
            body(rows3, dstr, None, z32, z1, uout, None, rv, dv, ev, dv2, u_sp, s_sp)

        fn = body_ns

    return functools.partial(
        pl.kernel, mesh=_mesh(),
        compiler_params=pltpu.CompilerParams(use_tc_tiling_on_sc=False),
        out_type=tuple(outs) if with_scalar else outs[0],
        scratch_types=scratch,
    )(fn)


def _scatter_zeros():
    z16 = jnp.zeros((_STR, 16), jnp.float32)
    z1 = jnp.zeros((_STR,), jnp.float32)
    return z16, z1


_BM = 2000
_BME = 2048


def _mm_kernel(x_ref, w_ref, b_ref, o_ref):
    o_ref[...] = (
        jnp.dot(x_ref[...], w_ref[...], preferred_element_type=jnp.float32)
        + b_ref[...]
    )


def _matmul_bias(x, w, b, bm=_BM):
    m, k = x.shape
    _, n = w.shape
    return pl.pallas_call(
        _mm_kernel,
        grid=(m // bm,),
        in_specs=[
            pl.BlockSpec((bm, k), lambda i: (i, 0)),
            pl.BlockSpec((k, n), lambda i: (0, 0)),
            pl.BlockSpec((1, n), lambda i: (0, 0)),
        ],
        out_specs=pl.BlockSpec((bm, n), lambda i: (i, 0)),
        out_shape=jax.ShapeDtypeStruct((m, n), jnp.float32),
    )(x, w, b.reshape(1, n))


def _qkvs_kernel(o_ref, wq, wk, wv, ws, bq, bk, bv, bs, q_ref, k_ref, v_ref, s_ref):
    xx = o_ref[...]
    q_ref[...] = jnp.dot(xx, wq[...], preferred_element_type=jnp.float32) + bq[...]
    k_ref[...] = jnp.dot(xx, wk[...], preferred_element_type=jnp.float32) + bk[...]
    v_ref[...] = jnp.dot(xx, wv[...], preferred_element_type=jnp.float32) + bv[...]
    s_ref[...] = jnp.dot(xx, ws[...], preferred_element_type=jnp.float32) + bs[...]


def _qkvs(o, wq, wk, wv, ws, bq, bk, bv, bs):
    n = o.shape[0]
    wspec = pl.BlockSpec((64, 64), lambda i: (0, 0))
    bspec = pl.BlockSpec((1, 64), lambda i: (0, 0))
    ospec = pl.BlockSpec((_BM, 64), lambda i: (i, 0))
    return pl.pallas_call(
        _qkvs_kernel,
        grid=(n // _BM,),
        in_specs=[ospec, wspec, wspec, wspec, wspec, bspec, bspec, bspec, bspec],
        out_specs=[ospec, ospec, ospec, ospec],
        out_shape=[jax.ShapeDtypeStruct((n, 64), jnp.float32)] * 4,
    )(o, wq, wk, wv, ws, bq.reshape(1, 64), bk.reshape(1, 64), bv.reshape(1, 64),
      bs.reshape(1, 64))


def _alpha_kernel(qg_ref, kg_ref, ee_ref, a_ref):
    prod = qg_ref[...] * (kg_ref[...] + ee_ref[...])
    a_ref[...] = jnp.sum(prod, axis=1) * np.float32(0.125)


def _alpha(qg, kg, ee):
    m = qg.shape[0]
    spec = pl.BlockSpec((_BME, 64), lambda i: (i, 0))
    return pl.pallas_call(
        _alpha_kernel,
        grid=(pl.cdiv(m, _BME),),
        in_specs=[spec, spec, spec],
        out_specs=pl.BlockSpec((_BME,), lambda i: (i,)),
        out_shape=jax.ShapeDtypeStruct((m,), jnp.float32),
    )(qg, kg, ee)


def _maxmerge_kernel(m2_ref, o_ref):
    o_ref[...] = jnp.maximum(m2_ref[0], m2_ref[1]).reshape(_NP, 1)


def _maxmerge(m2):
    return pl.pallas_call(
        _maxmerge_kernel,
        grid=(1,),
        in_specs=[pl.BlockSpec((_NC, _NP), lambda i: (0, 0))],
        out_specs=pl.BlockSpec((_NP, 1), lambda i: (0, 0)),
        out_shape=jax.ShapeDtypeStruct((_NP, 1), jnp.float32),
    )(m2)


def _msgex_kernel(vg_ref, ee_ref, a_ref, mg_ref, pex_ref, ex_ref):
    ex = jnp.exp(a_ref[...] - mg_ref[...])
    pex = (vg_ref[...] + ee_ref[...]) * ex[:, None]
    for q in range(4):
        pex_ref[q] = pex[:, 16 * q:16 * (q + 1)]
    ex_ref[...] = ex


def _msgex(vg, ee, alpha, mg):
    m = vg.shape[0]
    spec = pl.BlockSpec((_BME, 64), lambda i: (i, 0))
    spec1 = pl.BlockSpec((_BME,), lambda i: (i,))
    return pl.pallas_call(
        _msgex_kernel,
        grid=(pl.cdiv(m, _BME),),
        in_specs=[spec, spec, spec1, spec1],
        out_specs=[
            pl.BlockSpec((4, _BME, 16), lambda i: (0, i, 0)),
            spec1,
        ],
        out_shape=[
            jax.ShapeDtypeStruct((4, m, 16), jnp.float32),
            jax.ShapeDtypeStruct((m,), jnp.float32),
        ],
    )(vg, ee, alpha, mg)


def _combine_kernel(u_ref, s_ref, os_ref, o_ref):
    u = jnp.concatenate([u_ref[q] for q in range(4)], axis=1)
    o_ref[...] = u / (s_ref[...][:, None] + 1e-16) + os_ref[...]


def _combine(u4, sv, os_):
    n = os_.shape[0]
    spec64 = pl.BlockSpec((_BME, 64), lambda i: (i, 0))
    return pl.pallas_call(
        _combine_kernel,
        grid=(pl.cdiv(n, _BME),),
        in_specs=[
            pl.BlockSpec((4, _BME, 16), lambda i: (0, i, 0)),
            pl.BlockSpec((_BME,), lambda i: (i,)),
            spec64,
        ],
        out_specs=spec64,
        out_shape=jax.ShapeDtypeStruct((n, 64), jnp.float32),
    )(u4, sv, os_)


def _genm_kernel(og_ref, e_ref, m_ref):
    mm = jnp.maximum(og_ref[...] + e_ref[...], 0.0) + 1e-7
    for q in range(4):
        m_ref[q] = mm[:, 16 * q:16 * (q + 1)]


def _genm(og, e):
    m = og.shape[0]
    spec = pl.BlockSpec((_BM, 64), lambda i: (i, 0))
    return pl.pallas_call(
        _genm_kernel,
        grid=(m // _BM,),
        in_specs=[spec, spec],
        out_specs=pl.BlockSpec((4, _BM, 16), lambda i: (0, i, 0)),
        out_shape=jax.ShapeDtypeStruct((4, m, 16), jnp.float32),
    )(og, e)


def _genout_kernel(a_ref, o_ref, w_ref, b_ref, out_ref):
    agg = jnp.concatenate([a_ref[q] for q in range(4)], axis=1)
    t = agg + o_ref[...]
    out_ref[...] = (
        jnp.dot(t, w_ref[...], preferred_element_type=jnp.float32) + b_ref[...]
    )


def _genout(a4, o, w, b):
    n = o.shape[0]
    spec64 = pl.BlockSpec((_BM, 64), lambda i: (i, 0))
    return pl.pallas_call(
        _genout_kernel,
        grid=(n // _BM,),
        in_specs=[
            pl.BlockSpec((4, _BM, 16), lambda i: (0, i, 0)),
            spec64,
            pl.BlockSpec((64, 64), lambda i: (0, 0)),
            pl.BlockSpec((1, 64), lambda i: (0, 0)),
        ],
        out_specs=spec64,
        out_shape=jax.ShapeDtypeStruct((n, 64), jnp.float32),
    )(a4, o, w, b.reshape(1, 64))


def _glob_kernel(o_ref, b_ref, sums_ref, cnt_ref):
    @pl.when(pl.program_id(0) == 0)
    def _():
        sums_ref[...] = jnp.zeros_like(sums_ref)
        cnt_ref[...] = jnp.zeros_like(cnt_ref)

    iot = lax.broadcasted_iota(jnp.int32, (128, _BM), 0)
    oh = (iot == b_ref[0]).astype(jnp.float32)
    sums_ref[...] += jnp.dot(oh, o_ref[...], preferred_element_type=jnp.float32)
    cnt_ref[...] += jnp.sum(oh, axis=1, keepdims=True)


def _glob(o, batch2d):
    n = o.shape[0]
    return pl.pallas_call(
        _glob_kernel,
        grid=(n // _BM,),
        in_specs=[
            pl.BlockSpec((_BM, 64), lambda i: (i, 0)),
            pl.BlockSpec((1, 1, _BM), lambda i: (i, 0, 0)),
        ],
        out_specs=[
            pl.BlockSpec((128, 64), lambda i: (0, 0)),
            pl.BlockSpec((128, 1), lambda i: (0, 0)),
        ],
        out_shape=[
            jax.ShapeDtypeStruct((128, 64), jnp.float32),
            jax.ShapeDtypeStruct((128, 1), jnp.float32),
        ],
    )(o, batch2d)


def _leaky(h):
    return jnp.where(h > 0, h, 0.01 * h)


def _headg_kernel(sums_ref, cnt_ref, w1s, b1s, w2s, b2s, w1r, b1r, w2r, b2r,
                  stop_ref, rew_ref):
    glob = sums_ref[...] / jnp.maximum(cnt_ref[...], 1.0)
    h1 = _leaky(jnp.dot(glob, w1s[...], preferred_element_type=jnp.float32) + b1s[...])
    stop_ref[...] = jnp.dot(h1, w2s[...], preferred_element_type=jnp.float32) + b2s[...]
    h2 = _leaky(jnp.dot(glob, w1r[...], preferred_element_type=jnp.float32) + b1r[...])
    rew_ref[...] = jnp.dot(h2, w2r[...], preferred_element_type=jnp.float32) + b2r[...]


def _headg(sums, cnt, p):
    full = lambda shp: pl.BlockSpec(shp, lambda i: tuple(0 for _ in shp))
    return pl.pallas_call(
        _headg_kernel,
        grid=(1,),
        in_specs=[full((128, 64)), full((128, 1)),
                  full((64, 64)), full((1, 64)), full((64, 1)), full((1, 1)),
                  full((64, 64)), full((1, 64)), full((64, 1)), full((1, 1))],
        out_specs=[full((128, 1)), full((128, 1))],
        out_shape=[jax.ShapeDtypeStruct((128, 1), jnp.float32)] * 2,
    )(sums, cnt,
      p['stop_W1'], p['stop_b1'].reshape(1, 64), p['stop_W2'], p['stop_b2'].reshape(1, 1),
      p['reward_W1'], p['reward_b1'].reshape(1, 64), p['reward_W2'], p['reward_b2'].reshape(1, 1))


def _head_kernel(a_ref, w1, b1, w2, b2, o_ref):
    h1 = _leaky(jnp.dot(a_ref[...], w1[...], preferred_element_type=jnp.float32) + b1[...])
    o_ref[...] = jnp.dot(h1, w2[...], preferred_element_type=jnp.float32) + b2[...]


def _head_pair_kernel(a_ref, b_ref, w1, b1, w2, b2, o_ref):
    h = a_ref[...] + b_ref[...]
    h1 = _leaky(jnp.dot(h, w1[...], preferred_element_type=jnp.float32) + b1[...])
    o_ref[...] = jnp.dot(h1, w2[...], preferred_element_type=jnp.float32) + b2[...]


def _head(a, w1, b1, w2, b2, b=None):
    m = a.shape[0]
    nl = w2.shape[1]
    spec = pl.BlockSpec((_BM, 64), lambda i: (i, 0))
    wspecs = [
        pl.BlockSpec((64, 64), lambda i: (0, 0)),
        pl.BlockSpec((1, 64), lambda i: (0, 0)),
        pl.BlockSpec((64, nl), lambda i: (0, 0)),
        pl.BlockSpec((1, nl), lambda i: (0, 0)),
    ]
    args = [a] if b is None else [a, b]
    return pl.pallas_call(
        _head_kernel if b is None else _head_pair_kernel,
        grid=(m // _BM,),
        in_specs=[spec] * len(args) + wspecs,
        out_specs=pl.BlockSpec((_BM, nl), lambda i: (i, 0)),
        out_shape=jax.ShapeDtypeStruct((m, nl), jnp.float32),
    )(*args, w1, b1.reshape(1, 64), w2, b2.reshape(1, nl))


def kernel(x, edge_index, edge_attr, batch, non_edge_index, params):
    p = params
    n = x.shape[0]
    e_cnt = edge_index.shape[1]
    src = edge_index[0]
    dst = edge_index[1]
    z16, z1 = _scatter_zeros()

    gather_n = _sc_gather(n, 64, e_cnt)
    gather_qkv = _sc_gather_qkv(n, e_cnt)
    gather_m = _sc_gather(_NP, 1, e_cnt)
    segmax = _sc_segmax(e_cnt)
    scat_ex = _sc_scatter(e_cnt, True)
    scat_nx = _sc_scatter(e_cnt, False)

    o = _matmul_bias(x, p['x2h_W'], p['x2h_b'])
    e = _matmul_bias(edge_attr, p['e2h_W'], p['e2h_b'])

    for i in range(6):
        q, k, v, os_ = _qkvs(
            o, p['tc_Wq'][i], p['tc_Wk'][i], p['tc_Wv'][i], p['tc_Ws'][i],
            p['tc_bq'][i], p['tc_bk'][i], p['tc_bv'][i], p['tc_bs'][i])
        ee = _matmul_bias(e, p['tc_We'][i], p['tc_be'][i])
        qg, kg, vg = gather_qkv(q, k, v, src, dst)
        alpha = _alpha(qg, kg, ee)
        m2 = segmax(alpha, dst)
        mtab = _maxmerge(m2)
        mg = gather_m(mtab, dst).reshape(e_cnt)
        pex4, exv = _msgex(vg, ee, alpha, mg)
        u4, sv = scat_ex(pex4, dst, exv, z16, z1)
        o = _combine(u4[:, :n], sv[:n], os_)
        og = gather_n(o, src)
        m4 = _genm(og, e)
        a4 = scat_nx(m4, dst, z16, z1)
        o = _genout(a4[:, :n], o, p['gen_W'][i], p['gen_b'][i])

    sums, cnt = _glob(o, batch.reshape(n // _BM, 1, _BM))
    stop_logits, reward = _headg(sums, cnt, p)
    add_node_logits = _head(o, p['add_node_W1'], p['add_node_b1'],
                            p['add_node_W2'], p['add_node_b2'])

    ne_cnt = non_edge_index.shape[1]
    gather_ne = _sc_gather(n, 64, ne_cnt)
    oa = gather_ne(o, non_edge_index[0])
    ob = gather_ne(o, non_edge_index[1])
    add_edge_logits = _head(oa, p['add_edge_W1'], p['add_edge_b1'],
                            p['add_edge_W2'], p['add_edge_b2'], b=ob)

    er = edge_index[0, ::2]
    ec = edge_index[1, ::2]
    oc = gather_ne(o, er)
    od = gather_ne(o, ec)
    add_edge_attr_logits = _head(oc, p['add_edge_attr_W1'], p['add_edge_attr_b1'],
                                 p['add_edge_attr_W2'], p['add_edge_attr_b2'], b=od)

    return (stop_logits, add_node_logits, add_edge_logits, add_edge_attr_logits,
            reward)
